# baseline (speedup 1.0000x reference)
import functools

import jax
import jax.numpy as jnp
from jax import lax
from jax.experimental import pallas as pl
from jax.experimental.pallas import tpu as pltpu
from jax.experimental.pallas import tpu_sc as plsc

N = 10000
NP = 10240
E = 320000
F0 = 128
NG = 64
K = 30
NC = 2
NS = 16
NW = NC * NS
CPW = 80
EPT = CPW * 128
EPAD = NW * EPT
RPT = NP // NS
DUMMY = N

def _mesh():
    return plsc.VectorSubcoreMesh(core_axis_name="c", subcore_axis_name="s",
                                  num_cores=NC, num_subcores=NS)


@functools.cache
def _sc_prep_kernel():
    return functools.partial(
        pl.kernel,
        mesh=_mesh(),
        compiler_params=pltpu.CompilerParams(needs_layout_passes=False),
        out_type=[
            jax.ShapeDtypeStruct((NC, NP), jnp.float32),
            jax.ShapeDtypeStruct((NW, CPW, 128), jnp.int32),
            jax.ShapeDtypeStruct((128,), jnp.int32),
        ],
        scratch_types=[
            pltpu.VMEM((CPW, 128), jnp.int32),
            pltpu.VMEM((CPW, 128), jnp.int32),
            pltpu.VMEM((CPW, 128), jnp.int32),
            pltpu.VMEM((NP,), jnp.float32),
            pltpu.VMEM_SHARED((NS, NP), jnp.float32),
            pltpu.VMEM((NS, RPT), jnp.float32),
            pltpu.VMEM((RPT,), jnp.float32),
            pltpu.VMEM((N,), jnp.int32),
            pltpu.VMEM((64,), jnp.int32),
            pltpu.VMEM((128,), jnp.int32),
        ],
    )(_sc_prep_body)


def _sc_prep_body(src_hbm, dst_hbm, batch_hbm, deg_out, srcp_out, bounds_out,
             src_v, dst_v, srcp_v, deg_v, shared_deg, part_v, red_v,
             batch_v, counts_v, bounds_v):
    cid = lax.axis_index("c")
    sid = lax.axis_index("s")
    wid = sid * NC + cid

    zf = jnp.zeros((16,), jnp.float32)

    def _zero(i, _):
        deg_v[pl.ds(i * 16, 16)] = zf
        return 0
    lax.fori_loop(0, NP // 16, _zero, 0)

    pltpu.sync_copy(src_hbm.at[wid], src_v)
    pltpu.sync_copy(dst_hbm.at[wid], dst_v)

    dummy16 = jnp.full((16,), DUMMY, jnp.int32)

    def _edges(j, _):
        for k in range(8):
            s = src_v[j, pl.ds(k * 16, 16)]
            d = dst_v[j, pl.ds(k * 16, 16)]
            m = s != d
            plsc.addupdate_scatter(
                deg_v, [d], jnp.where(m, 1.0, 0.0).astype(jnp.float32))
            srcp_v[j, pl.ds(k * 16, 16)] = jnp.where(m, s, dummy16)
        return 0
    lax.fori_loop(0, CPW, _edges, 0)
    pltpu.sync_copy(srcp_v, srcp_out.at[wid])

    pltpu.sync_copy(deg_v, shared_deg.at[sid])
    plsc.subcore_barrier()
    pltpu.sync_copy(shared_deg.at[:, pl.ds(sid * RPT, RPT)], part_v)

    def _red(i, _):
        acc = jnp.zeros((16,), jnp.float32)
        for k in range(NS):
            acc = acc + part_v[k, pl.ds(i * 16, 16)]
        red_v[pl.ds(i * 16, 16)] = acc
        return 0
    lax.fori_loop(0, RPT // 16, _red, 0)
    pltpu.sync_copy(red_v, deg_out.at[cid, pl.ds(sid * RPT, RPT)])

    @pl.when(jnp.logical_and(cid == 0, sid == 0))
    def _bounds():
        pltpu.sync_copy(batch_hbm, batch_v)
        zi = jnp.zeros((16,), jnp.int32)
        for i in range(4):
            counts_v[pl.ds(i * 16, 16)] = zi
        ones_i = jnp.ones((16,), jnp.int32)

        def _cnt(i, _):
            b = batch_v[pl.ds(i * 16, 16)]
            plsc.addupdate_scatter(counts_v, [b], ones_i)
            return 0
        lax.fori_loop(0, N // 16, _cnt, 0)

        carry = jnp.int32(0)
        for g in range(4):
            c = counts_v[pl.ds(g * 16, 16)]
            cs = plsc.cumsum(c)
            bounds_v[pl.ds(g * 16, 16)] = carry + cs - c
            bounds_v[pl.ds(64 + g * 16, 16)] = carry + cs
            carry = carry + jnp.sum(c)
        pltpu.sync_copy(bounds_v, bounds_out)


@functools.cache
def _make_scatter(F):
    @functools.partial(
        pl.kernel,
        mesh=_mesh(),
        compiler_params=pltpu.CompilerParams(needs_layout_passes=False,
                                             use_tc_tiling_on_sc=False),
        out_type=jax.ShapeDtypeStruct((NC, NP, F), jnp.float32),
        scratch_types=[
            pltpu.VMEM((CPW, 128), jnp.int32),
            pltpu.VMEM((CPW, 128), jnp.int32),
            pltpu.VMEM((128, F), jnp.float32),
            pltpu.VMEM((128, F), jnp.float32),
            pltpu.VMEM((RPT, F), jnp.float32),
            pltpu.VMEM_SHARED((NP, F), jnp.float32),
            pltpu.SemaphoreType.DMA,
            pltpu.SemaphoreType.DMA,
        ],
    )
    def _scatter(hs_hbm, srcp_hbm, dst_hbm, acc_out,
                 sidx, didx, rows0, rows1, zbuf, acc_sh, semg0, semg1):
        cid = lax.axis_index("c")
        sid = lax.axis_index("s")
        wid = sid * NC + cid

        zf = jnp.zeros((16,), jnp.float32)

        def _zero(i, _):
            for k in range(F // 16):
                zbuf[i, pl.ds(k * 16, 16)] = zf
            return 0
        lax.fori_loop(0, RPT, _zero, 0)
        pltpu.sync_copy(zbuf, acc_sh.at[pl.ds(sid * RPT, RPT), :])

        pltpu.sync_copy(srcp_hbm.at[wid], sidx)
        pltpu.sync_copy(dst_hbm.at[wid], didx)
        plsc.subcore_barrier()

        npair = CPW // 2
        pltpu.async_copy(hs_hbm.at[sidx.at[0]], rows0, semg0)

        def _edge_pair(j2, _):
            j0 = 2 * j2
            pltpu.make_async_copy(hs_hbm.at[sidx.at[j0]], rows0, semg0).wait()
            pltpu.async_copy(hs_hbm.at[sidx.at[j0 + 1]], rows1, semg1)
            pltpu.sync_copy(rows0, acc_sh.at[didx.at[j0]], add=True)
            pltpu.make_async_copy(
                hs_hbm.at[sidx.at[j0 + 1]], rows1, semg1).wait()

            @pl.when(j2 + 1 < npair)
            def _prefetch():
                pltpu.async_copy(hs_hbm.at[sidx.at[j0 + 2]], rows0, semg0)

            pltpu.sync_copy(rows1, acc_sh.at[didx.at[j0 + 1]], add=True)
            return 0
        lax.fori_loop(0, npair, _edge_pair, 0)

        plsc.subcore_barrier()
        pltpu.sync_copy(acc_sh.at[pl.ds(sid * RPT, RPT), :], zbuf)
        pltpu.sync_copy(zbuf, acc_out.at[cid, pl.ds(sid * RPT, RPT), :])

    return _scatter


GPT = NG // NW
_NEG_INF = float("-inf")
_IMAX = 2147483647


@functools.cache
def _sc_sortpool_kernel():
    return functools.partial(
        pl.kernel,
        mesh=_mesh(),
        compiler_params=pltpu.CompilerParams(needs_layout_passes=False),
        out_type=jax.ShapeDtypeStruct((NG * 32, 128), jnp.float32),
        scratch_types=[
            pltpu.VMEM((NP,), jnp.float32),
            pltpu.VMEM((160,), jnp.int32),
            pltpu.VMEM((32,), jnp.int32),
            pltpu.VMEM((32, 128), jnp.float32),
            pltpu.SemaphoreType.DMA,
        ],
    )(_sc_sortpool_body)


def _sc_sortpool_body(keys_hbm, bounds_hbm, xc_hbm, sel_out,
                 keys_v, bounds_v, idx_buf, rows, sem):
    cid = lax.axis_index("c")
    sid = lax.axis_index("s")
    wid = sid * NC + cid
    pltpu.sync_copy(keys_hbm, keys_v)
    pltpu.sync_copy(bounds_hbm, bounds_v.at[pl.ds(0, 128)])

    lane = jnp.arange(16, dtype=jnp.int32)
    neg16 = jnp.full((16,), _NEG_INF, jnp.float32)
    dummy16 = jnp.full((16,), DUMMY, jnp.int32)

    for gi in range(GPT):
        g = wid * GPT + gi
        start = bounds_v[pl.ds(g, 16)][0]
        end = bounds_v[pl.ds(64 + g, 16)][0]
        c_lo = start // 16
        c_hi = (end + 15) // 16

        res = [dummy16, dummy16]
        for r in range(K):
            def _scan(c, carry):
                m_v, i_v = carry
                base = c * 16
                kv = keys_v[pl.ds(base, 16)]
                gidx = base + lane
                valid = jnp.logical_and(gidx >= start, gidx < end)
                kv = jnp.where(valid, kv, neg16)
                upd = kv > m_v
                return jnp.where(upd, kv, m_v), jnp.where(upd, gidx, i_v)

            m_v, i_v = lax.fori_loop(
                c_lo, c_hi, _scan,
                (neg16, jnp.zeros((16,), jnp.int32)))
            m = jnp.max(m_v)
            idx = jnp.min(jnp.where(m_v == m, i_v, _IMAX))
            is_valid = m > _NEG_INF
            idx_final = jnp.where(is_valid, idx, DUMMY)
            plsc.store_scatter(
                keys_v, [jnp.full((16,), idx, jnp.int32)], neg16,
                mask=jnp.logical_and(lane == 0, is_valid))
            q, sl = divmod(r, 16)
            res[q] = jnp.where(lane == sl, idx_final, res[q])

        idx_buf[pl.ds(0, 16)] = res[0]
        idx_buf[pl.ds(16, 16)] = res[1]
        pltpu.async_copy(xc_hbm.at[idx_buf], rows, sem).wait()
        pltpu.sync_copy(rows, sel_out.at[pl.ds(g * 32, 32), :])


_BR = 1024


def _tc_prep(deg3, x_pad, W1):
    def body(deg_ref, x_ref, w_ref, dinv_ref, hs_ref):
        deg = deg_ref[0] + deg_ref[1]
        dinv = jnp.where(deg > 0, lax.rsqrt(deg), 0.0)
        dinv_ref[...] = dinv
        h = jnp.dot(x_ref[...], w_ref[...],
                    preferred_element_type=jnp.float32)
        hs_ref[...] = dinv * h

    return pl.pallas_call(
        body,
        grid=(NP // _BR,),
        in_specs=[
            pl.BlockSpec((NC, _BR, 1), lambda i: (0, i, 0)),
            pl.BlockSpec((_BR, F0), lambda i: (i, 0)),
            pl.BlockSpec((F0, 32), lambda i: (0, 0)),
        ],
        out_specs=[
            pl.BlockSpec((_BR, 1), lambda i: (i, 0)),
            pl.BlockSpec((_BR, 32), lambda i: (i, 0)),
        ],
        out_shape=[
            jax.ShapeDtypeStruct((NP, 1), jnp.float32),
            jax.ShapeDtypeStruct((NP, 32), jnp.float32),
        ],
    )(deg3, x_pad, W1)


def _tc_layer(acc, hs, dinv, b, Wn, Fin, Fn):
    def body(acc_ref, hs_ref, dinv_ref, b_ref, wn_ref, x_ref, hn_ref):
        a = acc_ref[0] + acc_ref[1] + hs_ref[...]
        dinv = dinv_ref[...]
        xv = jnp.tanh(dinv * a + b_ref[...])
        xv = jnp.where(dinv > 0, xv, 0.0)
        x_ref[...] = xv
        hn_ref[...] = dinv * jnp.dot(xv, wn_ref[...],
                                     preferred_element_type=jnp.float32)

    return pl.pallas_call(
        body,
        grid=(NP // _BR,),
        in_specs=[
            pl.BlockSpec((NC, _BR, Fin), lambda i: (0, i, 0)),
            pl.BlockSpec((_BR, Fin), lambda i: (i, 0)),
            pl.BlockSpec((_BR, 1), lambda i: (i, 0)),
            pl.BlockSpec((1, Fin), lambda i: (0, 0)),
            pl.BlockSpec((Fin, Fn), lambda i: (0, 0)),
        ],
        out_specs=[
            pl.BlockSpec((_BR, Fin), lambda i: (i, 0)),
            pl.BlockSpec((_BR, Fn), lambda i: (i, 0)),
        ],
        out_shape=[
            jax.ShapeDtypeStruct((NP, Fin), jnp.float32),
            jax.ShapeDtypeStruct((NP, Fn), jnp.float32),
        ],
    )(acc, hs, dinv, b, Wn)


def _tc_layer_last(acc, hs, dinv, b, Fin):
    def body(acc_ref, hs_ref, dinv_ref, b_ref, x_ref):
        a = acc_ref[0] + acc_ref[1] + hs_ref[...]
        dinv = dinv_ref[...]
        xv = jnp.tanh(dinv * a + b_ref[...])
        x_ref[...] = jnp.where(dinv > 0, xv, 0.0)

    return pl.pallas_call(
        body,
        grid=(NP // _BR,),
        in_specs=[
            pl.BlockSpec((NC, _BR, Fin), lambda i: (0, i, 0)),
            pl.BlockSpec((_BR, Fin), lambda i: (i, 0)),
            pl.BlockSpec((_BR, 1), lambda i: (i, 0)),
            pl.BlockSpec((1, Fin), lambda i: (0, 0)),
        ],
        out_specs=pl.BlockSpec((_BR, Fin), lambda i: (i, 0)),
        out_shape=jax.ShapeDtypeStruct((NP, Fin), jnp.float32),
    )(acc, hs, dinv, b)


def _tc_head1(sel, W5p, b5):
    def body(s_ref, w_ref, b_ref, o_ref):
        o_ref[...] = jnp.maximum(
            jnp.dot(s_ref[...], w_ref[...],
                    preferred_element_type=jnp.float32) + b_ref[...], 0.0)

    return pl.pallas_call(
        body,
        out_shape=jax.ShapeDtypeStruct((NG * 32, 16), jnp.float32),
    )(sel, W5p, b5)


def _tc_head2(hp, W6, b6, fc1p, fc1b, fc2, fc2b):
    def body(hp_ref, w6_ref, b6_ref, f1_ref, f1b_ref, f2_ref, f2b_ref, o_ref):
        hp = hp_ref[...]
        pooled = jnp.concatenate(
            [jnp.maximum(hp[:, 32 * j:32 * j + 16],
                         hp[:, 32 * j + 16:32 * j + 32]) for j in range(15)],
            axis=1)
        w6 = w6_ref[...]
        b6 = b6_ref[...]
        h6 = jnp.concatenate(
            [jnp.maximum(
                jnp.dot(pooled[:, 16 * l:16 * l + 80], w6,
                        preferred_element_type=jnp.float32) + b6, 0.0)
             for l in range(11)], axis=1)
        h = jnp.maximum(
            jnp.dot(h6, f1_ref[...],
                    preferred_element_type=jnp.float32) + f1b_ref[...], 0.0)
        logits = jnp.dot(h, f2_ref[...],
                         preferred_element_type=jnp.float32) + f2b_ref[...]
        m = jnp.max(logits, axis=1, keepdims=True)
        lse = jnp.log(jnp.sum(jnp.exp(logits - m), axis=1, keepdims=True)) + m
        o_ref[...] = logits - lse

    return pl.pallas_call(
        body,
        out_shape=jax.ShapeDtypeStruct((NG, 10), jnp.float32),
    )(hp, W6, b6, fc1p, fc1b, fc2, fc2b)


def kernel(x, edge_index, batch, W1, b1, W2, b2, W3, b3, W4, b4,
           conv5_w, conv5_b, conv6_w, conv6_b, fc1_w, fc1_b, fc2_w, fc2_b):
    src = jnp.pad(edge_index[0], (0, EPAD - E)).reshape(NW, CPW, 128)
    dst = jnp.pad(edge_index[1], (0, EPAD - E)).reshape(NW, CPW, 128)
    x_pad = jnp.pad(x, ((0, NP - N), (0, 0)))

    deg_part, srcp, bounds = _sc_prep_kernel()(src, dst, batch)
    dinv, h1s = _tc_prep(deg_part.reshape(NC, NP, 1), x_pad, W1)

    _scatter32 = _make_scatter(32)
    acc1 = _scatter32(h1s, srcp, dst)
    x1, h2s = _tc_layer(acc1, h1s, dinv, b1.reshape(1, 32), W2, 32, 32)
    acc2 = _scatter32(h2s, srcp, dst)
    x2, h3s = _tc_layer(acc2, h2s, dinv, b2.reshape(1, 32), W3, 32, 32)
    acc3 = _scatter32(h3s, srcp, dst)
    W4p = jnp.pad(W4, ((0, 0), (0, 15)))
    x3, h4s = _tc_layer(acc3, h3s, dinv, b3.reshape(1, 32), W4p, 32, 16)
    acc4 = _make_scatter(16)(h4s, srcp, dst)
    b4p = jnp.pad(b4, (0, 15)).reshape(1, 16)
    x4 = _tc_layer_last(acc4, h4s, dinv, b4p, 16)

    xc = jnp.concatenate(
        [x1, x2, x3, x4[:, :1], jnp.zeros((NP, 31), jnp.float32)], axis=1)
    keys = x4[:, 0]

    sel = _sc_sortpool_kernel()(keys, bounds, xc)

    W5p = jnp.pad(conv5_w[:, 0, :].T, ((0, 31), (0, 0)))
    c5 = _tc_head1(sel, W5p, conv5_b.reshape(1, 16))
    hp = c5.reshape(NG, 32, 16)[:, :K, :].reshape(NG, K * 16)

    W6 = conv6_w.transpose(2, 1, 0).reshape(80, 32)
    fc1p = fc1_w.reshape(32, 11, 128).transpose(1, 0, 2).reshape(352, 128)
    return _tc_head2(hp, W6, conv6_b.reshape(1, 32), fc1p,
                     fc1_b.reshape(1, 128), fc2_w, fc2_b.reshape(1, 10))


# speedup vs baseline: 1.0733x; 1.0733x over previous
import functools

import jax
import jax.numpy as jnp
from jax import lax
from jax.experimental import pallas as pl
from jax.experimental.pallas import tpu as pltpu
from jax.experimental.pallas import tpu_sc as plsc

N = 10000
NP = 10240
E = 320000
F0 = 128
NG = 64
K = 30
NC = 2
NS = 16
NW = NC * NS
CPW = 80
EPT = CPW * 128
EPAD = NW * EPT
RPT = NP // NS
GRP = 4
DUMMY = N

def _mesh():
    return plsc.VectorSubcoreMesh(core_axis_name="c", subcore_axis_name="s",
                                  num_cores=NC, num_subcores=NS)


@functools.cache
def _sc_prep_kernel():
    return functools.partial(
        pl.kernel,
        mesh=_mesh(),
        compiler_params=pltpu.CompilerParams(needs_layout_passes=False),
        out_type=[
            jax.ShapeDtypeStruct((NC, NP), jnp.float32),
            jax.ShapeDtypeStruct((NW, CPW, 128), jnp.int32),
            jax.ShapeDtypeStruct((128,), jnp.int32),
        ],
        scratch_types=[
            pltpu.VMEM((CPW, 128), jnp.int32),
            pltpu.VMEM((CPW, 128), jnp.int32),
            pltpu.VMEM((CPW, 128), jnp.int32),
            pltpu.VMEM((NP,), jnp.float32),
            pltpu.VMEM_SHARED((NS, NP), jnp.float32),
            pltpu.VMEM((NS, RPT), jnp.float32),
            pltpu.VMEM((RPT,), jnp.float32),
            pltpu.VMEM((N,), jnp.int32),
            pltpu.VMEM((64,), jnp.int32),
            pltpu.VMEM((128,), jnp.int32),
        ],
    )(_sc_prep_body)


def _sc_prep_body(src_hbm, dst_hbm, batch_hbm, deg_out, srcp_out, bounds_out,
             src_v, dst_v, srcp_v, deg_v, shared_deg, part_v, red_v,
             batch_v, counts_v, bounds_v):
    cid = lax.axis_index("c")
    sid = lax.axis_index("s")
    wid = sid * NC + cid

    zf = jnp.zeros((16,), jnp.float32)

    def _zero(i, _):
        deg_v[pl.ds(i * 16, 16)] = zf
        return 0
    lax.fori_loop(0, NP // 16, _zero, 0)

    pltpu.sync_copy(src_hbm.at[wid], src_v)
    pltpu.sync_copy(dst_hbm.at[wid], dst_v)

    dummy16 = jnp.full((16,), DUMMY, jnp.int32)

    def _edges(j, _):
        for k in range(8):
            s = src_v[j, pl.ds(k * 16, 16)]
            d = dst_v[j, pl.ds(k * 16, 16)]
            m = s != d
            plsc.addupdate_scatter(
                deg_v, [d], jnp.where(m, 1.0, 0.0).astype(jnp.float32))
            srcp_v[j, pl.ds(k * 16, 16)] = jnp.where(m, s, dummy16)
        return 0
    lax.fori_loop(0, CPW, _edges, 0)
    pltpu.sync_copy(srcp_v, srcp_out.at[wid])

    pltpu.sync_copy(deg_v, shared_deg.at[sid])
    plsc.subcore_barrier()
    pltpu.sync_copy(shared_deg.at[:, pl.ds(sid * RPT, RPT)], part_v)

    def _red(i, _):
        acc = jnp.zeros((16,), jnp.float32)
        for k in range(NS):
            acc = acc + part_v[k, pl.ds(i * 16, 16)]
        red_v[pl.ds(i * 16, 16)] = acc
        return 0
    lax.fori_loop(0, RPT // 16, _red, 0)
    pltpu.sync_copy(red_v, deg_out.at[cid, pl.ds(sid * RPT, RPT)])

    @pl.when(jnp.logical_and(cid == 0, sid == 0))
    def _bounds():
        pltpu.sync_copy(batch_hbm, batch_v)
        zi = jnp.zeros((16,), jnp.int32)
        for i in range(4):
            counts_v[pl.ds(i * 16, 16)] = zi
        ones_i = jnp.ones((16,), jnp.int32)

        def _cnt(i, _):
            b = batch_v[pl.ds(i * 16, 16)]
            plsc.addupdate_scatter(counts_v, [b], ones_i)
            return 0
        lax.fori_loop(0, N // 16, _cnt, 0)

        carry = jnp.int32(0)
        for g in range(4):
            c = counts_v[pl.ds(g * 16, 16)]
            cs = plsc.cumsum(c)
            bounds_v[pl.ds(g * 16, 16)] = carry + cs - c
            bounds_v[pl.ds(64 + g * 16, 16)] = carry + cs
            carry = carry + jnp.sum(c)
        pltpu.sync_copy(bounds_v, bounds_out)


@functools.cache
def _make_scatter(F):
    @functools.partial(
        pl.kernel,
        mesh=_mesh(),
        compiler_params=pltpu.CompilerParams(needs_layout_passes=False,
                                             use_tc_tiling_on_sc=False),
        out_type=jax.ShapeDtypeStruct((NC, NP, F), jnp.float32),
        scratch_types=[
            pltpu.VMEM((CPW // GRP, GRP * 128), jnp.int32),
            pltpu.VMEM((CPW // GRP, GRP * 128), jnp.int32),
            pltpu.VMEM((GRP * 128, F), jnp.float32),
            pltpu.VMEM((RPT, F), jnp.float32),
            pltpu.VMEM_SHARED((NP, F), jnp.float32),
            pltpu.SemaphoreType.DMA,
        ],
    )
    def _scatter(hs_hbm, srcp_hbm, dst_hbm, acc_out,
                 sidx, didx, rows, zbuf, acc_sh, sem):
        cid = lax.axis_index("c")
        sid = lax.axis_index("s")
        wid = sid * NC + cid

        zf = jnp.zeros((16,), jnp.float32)

        def _zero(i, _):
            for k in range(F // 16):
                zbuf[i, pl.ds(k * 16, 16)] = zf
            return 0
        lax.fori_loop(0, RPT, _zero, 0)
        pltpu.sync_copy(zbuf, acc_sh.at[pl.ds(sid * RPT, RPT), :])

        pltpu.sync_copy(srcp_hbm.at[wid], sidx)
        pltpu.sync_copy(dst_hbm.at[wid], didx)
        plsc.subcore_barrier()

        def _edge_chunk(j, _):
            pltpu.async_copy(hs_hbm.at[sidx.at[j]], rows, sem).wait()
            pltpu.sync_copy(rows, acc_sh.at[didx.at[j]], add=True)
            return 0
        lax.fori_loop(0, CPW // GRP, _edge_chunk, 0)

        plsc.subcore_barrier()
        pltpu.sync_copy(acc_sh.at[pl.ds(sid * RPT, RPT), :], zbuf)
        pltpu.sync_copy(zbuf, acc_out.at[cid, pl.ds(sid * RPT, RPT), :])

    return _scatter


GPT = NG // NW
_NEG_INF = float("-inf")
_IMAX = 2147483647


@functools.cache
def _sc_sortpool_kernel():
    return functools.partial(
        pl.kernel,
        mesh=_mesh(),
        compiler_params=pltpu.CompilerParams(needs_layout_passes=False),
        out_type=jax.ShapeDtypeStruct((NG * 32, 128), jnp.float32),
        scratch_types=[
            pltpu.VMEM((NP,), jnp.float32),
            pltpu.VMEM((160,), jnp.int32),
            pltpu.VMEM((32,), jnp.int32),
            pltpu.VMEM((32, 128), jnp.float32),
            pltpu.SemaphoreType.DMA,
        ],
    )(_sc_sortpool_body)


def _sc_sortpool_body(keys_hbm, bounds_hbm, xc_hbm, sel_out,
                 keys_v, bounds_v, idx_buf, rows, sem):
    cid = lax.axis_index("c")
    sid = lax.axis_index("s")
    wid = sid * NC + cid
    pltpu.sync_copy(keys_hbm, keys_v)
    pltpu.sync_copy(bounds_hbm, bounds_v.at[pl.ds(0, 128)])

    lane = jnp.arange(16, dtype=jnp.int32)
    neg16 = jnp.full((16,), _NEG_INF, jnp.float32)
    dummy16 = jnp.full((16,), DUMMY, jnp.int32)

    for gi in range(GPT):
        g = wid * GPT + gi
        start = bounds_v[pl.ds(g, 16)][0]
        end = bounds_v[pl.ds(64 + g, 16)][0]
        c_lo = start // 16
        c_hi = (end + 15) // 16

        res = [dummy16, dummy16]
        for r in range(K):
            def _scan(c, carry):
                m_v, i_v = carry
                base = c * 16
                kv = keys_v[pl.ds(base, 16)]
                gidx = base + lane
                valid = jnp.logical_and(gidx >= start, gidx < end)
                kv = jnp.where(valid, kv, neg16)
                upd = kv > m_v
                return jnp.where(upd, kv, m_v), jnp.where(upd, gidx, i_v)

            m_v, i_v = lax.fori_loop(
                c_lo, c_hi, _scan,
                (neg16, jnp.zeros((16,), jnp.int32)))
            m = jnp.max(m_v)
            idx = jnp.min(jnp.where(m_v == m, i_v, _IMAX))
            is_valid = m > _NEG_INF
            idx_final = jnp.where(is_valid, idx, DUMMY)
            plsc.store_scatter(
                keys_v, [jnp.full((16,), idx, jnp.int32)], neg16,
                mask=jnp.logical_and(lane == 0, is_valid))
            q, sl = divmod(r, 16)
            res[q] = jnp.where(lane == sl, idx_final, res[q])

        idx_buf[pl.ds(0, 16)] = res[0]
        idx_buf[pl.ds(16, 16)] = res[1]
        pltpu.async_copy(xc_hbm.at[idx_buf], rows, sem).wait()
        pltpu.sync_copy(rows, sel_out.at[pl.ds(g * 32, 32), :])


_BR = 1024


def _tc_prep(deg3, x_pad, W1):
    def body(deg_ref, x_ref, w_ref, dinv_ref, hs_ref):
        deg = deg_ref[0] + deg_ref[1]
        dinv = jnp.where(deg > 0, lax.rsqrt(deg), 0.0)
        dinv_ref[...] = dinv
        h = jnp.dot(x_ref[...], w_ref[...],
                    preferred_element_type=jnp.float32)
        hs_ref[...] = dinv * h

    return pl.pallas_call(
        body,
        grid=(NP // _BR,),
        in_specs=[
            pl.BlockSpec((NC, _BR, 1), lambda i: (0, i, 0)),
            pl.BlockSpec((_BR, F0), lambda i: (i, 0)),
            pl.BlockSpec((F0, 32), lambda i: (0, 0)),
        ],
        out_specs=[
            pl.BlockSpec((_BR, 1), lambda i: (i, 0)),
            pl.BlockSpec((_BR, 32), lambda i: (i, 0)),
        ],
        out_shape=[
            jax.ShapeDtypeStruct((NP, 1), jnp.float32),
            jax.ShapeDtypeStruct((NP, 32), jnp.float32),
        ],
    )(deg3, x_pad, W1)


def _tc_layer(acc, hs, dinv, b, Wn, Fin, Fn):
    def body(acc_ref, hs_ref, dinv_ref, b_ref, wn_ref, x_ref, hn_ref):
        a = acc_ref[0] + acc_ref[1] + hs_ref[...]
        dinv = dinv_ref[...]
        xv = jnp.tanh(dinv * a + b_ref[...])
        xv = jnp.where(dinv > 0, xv, 0.0)
        x_ref[...] = xv
        hn_ref[...] = dinv * jnp.dot(xv, wn_ref[...],
                                     preferred_element_type=jnp.float32)

    return pl.pallas_call(
        body,
        grid=(NP // _BR,),
        in_specs=[
            pl.BlockSpec((NC, _BR, Fin), lambda i: (0, i, 0)),
            pl.BlockSpec((_BR, Fin), lambda i: (i, 0)),
            pl.BlockSpec((_BR, 1), lambda i: (i, 0)),
            pl.BlockSpec((1, Fin), lambda i: (0, 0)),
            pl.BlockSpec((Fin, Fn), lambda i: (0, 0)),
        ],
        out_specs=[
            pl.BlockSpec((_BR, Fin), lambda i: (i, 0)),
            pl.BlockSpec((_BR, Fn), lambda i: (i, 0)),
        ],
        out_shape=[
            jax.ShapeDtypeStruct((NP, Fin), jnp.float32),
            jax.ShapeDtypeStruct((NP, Fn), jnp.float32),
        ],
    )(acc, hs, dinv, b, Wn)


def _tc_layer_last(acc, hs, dinv, b, Fin):
    def body(acc_ref, hs_ref, dinv_ref, b_ref, x_ref):
        a = acc_ref[0] + acc_ref[1] + hs_ref[...]
        dinv = dinv_ref[...]
        xv = jnp.tanh(dinv * a + b_ref[...])
        x_ref[...] = jnp.where(dinv > 0, xv, 0.0)

    return pl.pallas_call(
        body,
        grid=(NP // _BR,),
        in_specs=[
            pl.BlockSpec((NC, _BR, Fin), lambda i: (0, i, 0)),
            pl.BlockSpec((_BR, Fin), lambda i: (i, 0)),
            pl.BlockSpec((_BR, 1), lambda i: (i, 0)),
            pl.BlockSpec((1, Fin), lambda i: (0, 0)),
        ],
        out_specs=pl.BlockSpec((_BR, Fin), lambda i: (i, 0)),
        out_shape=jax.ShapeDtypeStruct((NP, Fin), jnp.float32),
    )(acc, hs, dinv, b)


def _tc_head1(sel, W5p, b5):
    def body(s_ref, w_ref, b_ref, o_ref):
        o_ref[...] = jnp.maximum(
            jnp.dot(s_ref[...], w_ref[...],
                    preferred_element_type=jnp.float32) + b_ref[...], 0.0)

    return pl.pallas_call(
        body,
        out_shape=jax.ShapeDtypeStruct((NG * 32, 16), jnp.float32),
    )(sel, W5p, b5)


def _tc_head2(hp, W6, b6, fc1p, fc1b, fc2, fc2b):
    def body(hp_ref, w6_ref, b6_ref, f1_ref, f1b_ref, f2_ref, f2b_ref, o_ref):
        hp = hp_ref[...]
        pooled = jnp.concatenate(
            [jnp.maximum(hp[:, 32 * j:32 * j + 16],
                         hp[:, 32 * j + 16:32 * j + 32]) for j in range(15)],
            axis=1)
        w6 = w6_ref[...]
        b6 = b6_ref[...]
        h6 = jnp.concatenate(
            [jnp.maximum(
                jnp.dot(pooled[:, 16 * l:16 * l + 80], w6,
                        preferred_element_type=jnp.float32) + b6, 0.0)
             for l in range(11)], axis=1)
        h = jnp.maximum(
            jnp.dot(h6, f1_ref[...],
                    preferred_element_type=jnp.float32) + f1b_ref[...], 0.0)
        logits = jnp.dot(h, f2_ref[...],
                         preferred_element_type=jnp.float32) + f2b_ref[...]
        m = jnp.max(logits, axis=1, keepdims=True)
        lse = jnp.log(jnp.sum(jnp.exp(logits - m), axis=1, keepdims=True)) + m
        o_ref[...] = logits - lse

    return pl.pallas_call(
        body,
        out_shape=jax.ShapeDtypeStruct((NG, 10), jnp.float32),
    )(hp, W6, b6, fc1p, fc1b, fc2, fc2b)


def kernel(x, edge_index, batch, W1, b1, W2, b2, W3, b3, W4, b4,
           conv5_w, conv5_b, conv6_w, conv6_b, fc1_w, fc1_b, fc2_w, fc2_b):
    src = jnp.pad(edge_index[0], (0, EPAD - E)).reshape(NW, CPW, 128)
    dst = jnp.pad(edge_index[1], (0, EPAD - E)).reshape(NW, CPW, 128)
    x_pad = jnp.pad(x, ((0, NP - N), (0, 0)))

    deg_part, srcp, bounds = _sc_prep_kernel()(src, dst, batch)
    dinv, h1s = _tc_prep(deg_part.reshape(NC, NP, 1), x_pad, W1)

    srcp_g = srcp.reshape(NW, CPW // GRP, GRP * 128)
    dst_g = dst.reshape(NW, CPW // GRP, GRP * 128)
    _scatter32 = _make_scatter(32)
    acc1 = _scatter32(h1s, srcp_g, dst_g)
    x1, h2s = _tc_layer(acc1, h1s, dinv, b1.reshape(1, 32), W2, 32, 32)
    acc2 = _scatter32(h2s, srcp_g, dst_g)
    x2, h3s = _tc_layer(acc2, h2s, dinv, b2.reshape(1, 32), W3, 32, 32)
    acc3 = _scatter32(h3s, srcp_g, dst_g)
    W4p = jnp.pad(W4, ((0, 0), (0, 15)))
    x3, h4s = _tc_layer(acc3, h3s, dinv, b3.reshape(1, 32), W4p, 32, 16)
    acc4 = _make_scatter(16)(h4s, srcp_g, dst_g)
    b4p = jnp.pad(b4, (0, 15)).reshape(1, 16)
    x4 = _tc_layer_last(acc4, h4s, dinv, b4p, 16)

    xc = jnp.concatenate(
        [x1, x2, x3, x4[:, :1], jnp.zeros((NP, 31), jnp.float32)], axis=1)
    keys = x4[:, 0]

    sel = _sc_sortpool_kernel()(keys, bounds, xc)

    W5p = jnp.pad(conv5_w[:, 0, :].T, ((0, 31), (0, 0)))
    c5 = _tc_head1(sel, W5p, conv5_b.reshape(1, 16))
    hp = c5.reshape(NG, 32, 16)[:, :K, :].reshape(NG, K * 16)

    W6 = conv6_w.transpose(2, 1, 0).reshape(80, 32)
    fc1p = fc1_w.reshape(32, 11, 128).transpose(1, 0, 2).reshape(352, 128)
    return _tc_head2(hp, W6, conv6_b.reshape(1, 32), fc1p,
                     fc1_b.reshape(1, 128), fc2_w, fc2_b.reshape(1, 10))


# speedup vs baseline: 1.9223x; 1.7910x over previous
import functools

import jax
import jax.numpy as jnp
from jax import lax
from jax.experimental import pallas as pl
from jax.experimental.pallas import tpu as pltpu
from jax.experimental.pallas import tpu_sc as plsc

N = 10000
NP = 10240
E = 320000
F0 = 128
NG = 64
K = 30
NC = 2
NS = 16
NW = NC * NS
CPW = 80
EPT = CPW * 128
EPAD = NW * EPT
RPT = NP // NS
GRP = 4
DUMMY = N

def _mesh():
    return plsc.VectorSubcoreMesh(core_axis_name="c", subcore_axis_name="s",
                                  num_cores=NC, num_subcores=NS)


@functools.cache
def _sc_prep_kernel():
    return functools.partial(
        pl.kernel,
        mesh=_mesh(),
        compiler_params=pltpu.CompilerParams(needs_layout_passes=False),
        out_type=[
            jax.ShapeDtypeStruct((NC, NP), jnp.float32),
            jax.ShapeDtypeStruct((NW, CPW, 128), jnp.int32),
            jax.ShapeDtypeStruct((128,), jnp.int32),
        ],
        scratch_types=[
            pltpu.VMEM((CPW, 128), jnp.int32),
            pltpu.VMEM((CPW, 128), jnp.int32),
            pltpu.VMEM((CPW, 128), jnp.int32),
            pltpu.VMEM((NP,), jnp.float32),
            pltpu.VMEM_SHARED((NS, NP), jnp.float32),
            pltpu.VMEM((NS, RPT), jnp.float32),
            pltpu.VMEM((RPT,), jnp.float32),
            pltpu.VMEM((N,), jnp.int32),
            pltpu.VMEM((64,), jnp.int32),
            pltpu.VMEM((128,), jnp.int32),
        ],
    )(_sc_prep_body)


def _sc_prep_body(src_hbm, dst_hbm, batch_hbm, deg_out, srcp_out, bounds_out,
             src_v, dst_v, srcp_v, deg_v, shared_deg, part_v, red_v,
             batch_v, counts_v, bounds_v):
    cid = lax.axis_index("c")
    sid = lax.axis_index("s")
    wid = sid * NC + cid

    zf = jnp.zeros((16,), jnp.float32)

    def _zero(i, _):
        deg_v[pl.ds(i * 16, 16)] = zf
        return 0
    lax.fori_loop(0, NP // 16, _zero, 0)

    pltpu.sync_copy(src_hbm.at[wid], src_v)
    pltpu.sync_copy(dst_hbm.at[wid], dst_v)

    dummy16 = jnp.full((16,), DUMMY, jnp.int32)

    def _edges(j, _):
        for k in range(8):
            s = src_v[j, pl.ds(k * 16, 16)]
            d = dst_v[j, pl.ds(k * 16, 16)]
            m = s != d
            plsc.addupdate_scatter(
                deg_v, [d], jnp.where(m, 1.0, 0.0).astype(jnp.float32))
            srcp_v[j, pl.ds(k * 16, 16)] = jnp.where(m, s, dummy16)
        return 0
    lax.fori_loop(0, CPW, _edges, 0)
    pltpu.sync_copy(srcp_v, srcp_out.at[wid])

    pltpu.sync_copy(deg_v, shared_deg.at[sid])
    plsc.subcore_barrier()
    pltpu.sync_copy(shared_deg.at[:, pl.ds(sid * RPT, RPT)], part_v)

    def _red(i, _):
        acc = jnp.zeros((16,), jnp.float32)
        for k in range(NS):
            acc = acc + part_v[k, pl.ds(i * 16, 16)]
        red_v[pl.ds(i * 16, 16)] = acc
        return 0
    lax.fori_loop(0, RPT // 16, _red, 0)
    pltpu.sync_copy(red_v, deg_out.at[cid, pl.ds(sid * RPT, RPT)])

    @pl.when(jnp.logical_and(cid == 0, sid == 0))
    def _bounds():
        pltpu.sync_copy(batch_hbm, batch_v)
        zi = jnp.zeros((16,), jnp.int32)
        for i in range(4):
            counts_v[pl.ds(i * 16, 16)] = zi
        ones_i = jnp.ones((16,), jnp.int32)

        def _cnt(i, _):
            b = batch_v[pl.ds(i * 16, 16)]
            plsc.addupdate_scatter(counts_v, [b], ones_i)
            return 0
        lax.fori_loop(0, N // 16, _cnt, 0)

        carry = jnp.int32(0)
        for g in range(4):
            c = counts_v[pl.ds(g * 16, 16)]
            cs = plsc.cumsum(c)
            bounds_v[pl.ds(g * 16, 16)] = carry + cs - c
            bounds_v[pl.ds(64 + g * 16, 16)] = carry + cs
            carry = carry + jnp.sum(c)
        pltpu.sync_copy(bounds_v, bounds_out)


@functools.cache
def _make_scatter(F):
    @functools.partial(
        pl.kernel,
        mesh=_mesh(),
        compiler_params=pltpu.CompilerParams(needs_layout_passes=False,
                                             use_tc_tiling_on_sc=False),
        out_type=jax.ShapeDtypeStruct((NC, NP, F), jnp.float32),
        scratch_types=[
            pltpu.VMEM((CPW // GRP, GRP * 128), jnp.int32),
            pltpu.VMEM((CPW // GRP, GRP * 128), jnp.int32),
            pltpu.VMEM((GRP * 128, F), jnp.float32),
            pltpu.VMEM((RPT, F), jnp.float32),
            pltpu.VMEM_SHARED((NP, F), jnp.float32),
            pltpu.VMEM_SHARED((NP, F), jnp.float32),
            pltpu.SemaphoreType.DMA,
        ],
    )
    def _scatter(hs_hbm, srcp_hbm, dst_hbm, acc_out,
                 sidx, didx, rows, zbuf, acc_sh, hs_sh, sem):
        cid = lax.axis_index("c")
        sid = lax.axis_index("s")
        wid = sid * NC + cid

        zf = jnp.zeros((16,), jnp.float32)

        def _zero(i, _):
            for k in range(F // 16):
                zbuf[i, pl.ds(k * 16, 16)] = zf
            return 0
        lax.fori_loop(0, RPT, _zero, 0)
        pltpu.sync_copy(zbuf, acc_sh.at[pl.ds(sid * RPT, RPT), :])
        pltpu.sync_copy(hs_hbm.at[pl.ds(sid * RPT, RPT), :],
                        hs_sh.at[pl.ds(sid * RPT, RPT), :])

        pltpu.sync_copy(srcp_hbm.at[wid], sidx)
        pltpu.sync_copy(dst_hbm.at[wid], didx)
        plsc.subcore_barrier()

        def _edge_chunk(j, _):
            pltpu.async_copy(hs_sh.at[sidx.at[j]], rows, sem).wait()
            pltpu.sync_copy(rows, acc_sh.at[didx.at[j]], add=True)
            return 0
        lax.fori_loop(0, CPW // GRP, _edge_chunk, 0)

        plsc.subcore_barrier()
        pltpu.sync_copy(acc_sh.at[pl.ds(sid * RPT, RPT), :], zbuf)
        pltpu.sync_copy(zbuf, acc_out.at[cid, pl.ds(sid * RPT, RPT), :])

    return _scatter


GPT = NG // NW
_NEG_INF = float("-inf")
_IMAX = 2147483647


@functools.cache
def _sc_sortpool_kernel():
    return functools.partial(
        pl.kernel,
        mesh=_mesh(),
        compiler_params=pltpu.CompilerParams(needs_layout_passes=False),
        out_type=jax.ShapeDtypeStruct((NG * 32, 128), jnp.float32),
        scratch_types=[
            pltpu.VMEM((NP,), jnp.float32),
            pltpu.VMEM((160,), jnp.int32),
            pltpu.VMEM((32,), jnp.int32),
            pltpu.VMEM((32, 128), jnp.float32),
            pltpu.SemaphoreType.DMA,
        ],
    )(_sc_sortpool_body)


def _sc_sortpool_body(keys_hbm, bounds_hbm, xc_hbm, sel_out,
                 keys_v, bounds_v, idx_buf, rows, sem):
    cid = lax.axis_index("c")
    sid = lax.axis_index("s")
    wid = sid * NC + cid
    pltpu.sync_copy(keys_hbm, keys_v)
    pltpu.sync_copy(bounds_hbm, bounds_v.at[pl.ds(0, 128)])

    lane = jnp.arange(16, dtype=jnp.int32)
    neg16 = jnp.full((16,), _NEG_INF, jnp.float32)
    dummy16 = jnp.full((16,), DUMMY, jnp.int32)

    for gi in range(GPT):
        g = wid * GPT + gi
        start = bounds_v[pl.ds(g, 16)][0]
        end = bounds_v[pl.ds(64 + g, 16)][0]
        c_lo = start // 16
        c_hi = (end + 15) // 16

        res = [dummy16, dummy16]
        for r in range(K):
            def _scan(c, carry):
                m_v, i_v = carry
                base = c * 16
                kv = keys_v[pl.ds(base, 16)]
                gidx = base + lane
                valid = jnp.logical_and(gidx >= start, gidx < end)
                kv = jnp.where(valid, kv, neg16)
                upd = kv > m_v
                return jnp.where(upd, kv, m_v), jnp.where(upd, gidx, i_v)

            m_v, i_v = lax.fori_loop(
                c_lo, c_hi, _scan,
                (neg16, jnp.zeros((16,), jnp.int32)))
            m = jnp.max(m_v)
            idx = jnp.min(jnp.where(m_v == m, i_v, _IMAX))
            is_valid = m > _NEG_INF
            idx_final = jnp.where(is_valid, idx, DUMMY)
            plsc.store_scatter(
                keys_v, [jnp.full((16,), idx, jnp.int32)], neg16,
                mask=jnp.logical_and(lane == 0, is_valid))
            q, sl = divmod(r, 16)
            res[q] = jnp.where(lane == sl, idx_final, res[q])

        idx_buf[pl.ds(0, 16)] = res[0]
        idx_buf[pl.ds(16, 16)] = res[1]
        pltpu.async_copy(xc_hbm.at[idx_buf], rows, sem).wait()
        pltpu.sync_copy(rows, sel_out.at[pl.ds(g * 32, 32), :])


_BR = 1024


def _tc_prep(deg3, x_pad, W1):
    def body(deg_ref, x_ref, w_ref, dinv_ref, hs_ref):
        deg = deg_ref[0] + deg_ref[1]
        dinv = jnp.where(deg > 0, lax.rsqrt(deg), 0.0)
        dinv_ref[...] = dinv
        h = jnp.dot(x_ref[...], w_ref[...],
                    preferred_element_type=jnp.float32)
        hs_ref[...] = dinv * h

    return pl.pallas_call(
        body,
        grid=(NP // _BR,),
        in_specs=[
            pl.BlockSpec((NC, _BR, 1), lambda i: (0, i, 0)),
            pl.BlockSpec((_BR, F0), lambda i: (i, 0)),
            pl.BlockSpec((F0, 32), lambda i: (0, 0)),
        ],
        out_specs=[
            pl.BlockSpec((_BR, 1), lambda i: (i, 0)),
            pl.BlockSpec((_BR, 32), lambda i: (i, 0)),
        ],
        out_shape=[
            jax.ShapeDtypeStruct((NP, 1), jnp.float32),
            jax.ShapeDtypeStruct((NP, 32), jnp.float32),
        ],
    )(deg3, x_pad, W1)


def _tc_layer(acc, hs, dinv, b, Wn, Fin, Fn):
    def body(acc_ref, hs_ref, dinv_ref, b_ref, wn_ref, x_ref, hn_ref):
        a = acc_ref[0] + acc_ref[1] + hs_ref[...]
        dinv = dinv_ref[...]
        xv = jnp.tanh(dinv * a + b_ref[...])
        xv = jnp.where(dinv > 0, xv, 0.0)
        x_ref[...] = xv
        hn_ref[...] = dinv * jnp.dot(xv, wn_ref[...],
                                     preferred_element_type=jnp.float32)

    return pl.pallas_call(
        body,
        grid=(NP // _BR,),
        in_specs=[
            pl.BlockSpec((NC, _BR, Fin), lambda i: (0, i, 0)),
            pl.BlockSpec((_BR, Fin), lambda i: (i, 0)),
            pl.BlockSpec((_BR, 1), lambda i: (i, 0)),
            pl.BlockSpec((1, Fin), lambda i: (0, 0)),
            pl.BlockSpec((Fin, Fn), lambda i: (0, 0)),
        ],
        out_specs=[
            pl.BlockSpec((_BR, Fin), lambda i: (i, 0)),
            pl.BlockSpec((_BR, Fn), lambda i: (i, 0)),
        ],
        out_shape=[
            jax.ShapeDtypeStruct((NP, Fin), jnp.float32),
            jax.ShapeDtypeStruct((NP, Fn), jnp.float32),
        ],
    )(acc, hs, dinv, b, Wn)


def _tc_layer_last(acc, hs, dinv, b, Fin):
    def body(acc_ref, hs_ref, dinv_ref, b_ref, x_ref):
        a = acc_ref[0] + acc_ref[1] + hs_ref[...]
        dinv = dinv_ref[...]
        xv = jnp.tanh(dinv * a + b_ref[...])
        x_ref[...] = jnp.where(dinv > 0, xv, 0.0)

    return pl.pallas_call(
        body,
        grid=(NP // _BR,),
        in_specs=[
            pl.BlockSpec((NC, _BR, Fin), lambda i: (0, i, 0)),
            pl.BlockSpec((_BR, Fin), lambda i: (i, 0)),
            pl.BlockSpec((_BR, 1), lambda i: (i, 0)),
            pl.BlockSpec((1, Fin), lambda i: (0, 0)),
        ],
        out_specs=pl.BlockSpec((_BR, Fin), lambda i: (i, 0)),
        out_shape=jax.ShapeDtypeStruct((NP, Fin), jnp.float32),
    )(acc, hs, dinv, b)


def _tc_head1(sel, W5p, b5):
    def body(s_ref, w_ref, b_ref, o_ref):
        o_ref[...] = jnp.maximum(
            jnp.dot(s_ref[...], w_ref[...],
                    preferred_element_type=jnp.float32) + b_ref[...], 0.0)

    return pl.pallas_call(
        body,
        out_shape=jax.ShapeDtypeStruct((NG * 32, 16), jnp.float32),
    )(sel, W5p, b5)


def _tc_head2(hp, W6, b6, fc1p, fc1b, fc2, fc2b):
    def body(hp_ref, w6_ref, b6_ref, f1_ref, f1b_ref, f2_ref, f2b_ref, o_ref):
        hp = hp_ref[...]
        pooled = jnp.concatenate(
            [jnp.maximum(hp[:, 32 * j:32 * j + 16],
                         hp[:, 32 * j + 16:32 * j + 32]) for j in range(15)],
            axis=1)
        w6 = w6_ref[...]
        b6 = b6_ref[...]
        h6 = jnp.concatenate(
            [jnp.maximum(
                jnp.dot(pooled[:, 16 * l:16 * l + 80], w6,
                        preferred_element_type=jnp.float32) + b6, 0.0)
             for l in range(11)], axis=1)
        h = jnp.maximum(
            jnp.dot(h6, f1_ref[...],
                    preferred_element_type=jnp.float32) + f1b_ref[...], 0.0)
        logits = jnp.dot(h, f2_ref[...],
                         preferred_element_type=jnp.float32) + f2b_ref[...]
        m = jnp.max(logits, axis=1, keepdims=True)
        lse = jnp.log(jnp.sum(jnp.exp(logits - m), axis=1, keepdims=True)) + m
        o_ref[...] = logits - lse

    return pl.pallas_call(
        body,
        out_shape=jax.ShapeDtypeStruct((NG, 10), jnp.float32),
    )(hp, W6, b6, fc1p, fc1b, fc2, fc2b)


def kernel(x, edge_index, batch, W1, b1, W2, b2, W3, b3, W4, b4,
           conv5_w, conv5_b, conv6_w, conv6_b, fc1_w, fc1_b, fc2_w, fc2_b):
    src = jnp.pad(edge_index[0], (0, EPAD - E)).reshape(NW, CPW, 128)
    dst = jnp.pad(edge_index[1], (0, EPAD - E)).reshape(NW, CPW, 128)
    x_pad = jnp.pad(x, ((0, NP - N), (0, 0)))

    deg_part, srcp, bounds = _sc_prep_kernel()(src, dst, batch)
    dinv, h1s = _tc_prep(deg_part.reshape(NC, NP, 1), x_pad, W1)

    srcp_g = srcp.reshape(NW, CPW // GRP, GRP * 128)
    dst_g = dst.reshape(NW, CPW // GRP, GRP * 128)
    _scatter32 = _make_scatter(32)
    acc1 = _scatter32(h1s, srcp_g, dst_g)
    x1, h2s = _tc_layer(acc1, h1s, dinv, b1.reshape(1, 32), W2, 32, 32)
    acc2 = _scatter32(h2s, srcp_g, dst_g)
    x2, h3s = _tc_layer(acc2, h2s, dinv, b2.reshape(1, 32), W3, 32, 32)
    acc3 = _scatter32(h3s, srcp_g, dst_g)
    W4p = jnp.pad(W4, ((0, 0), (0, 15)))
    x3, h4s = _tc_layer(acc3, h3s, dinv, b3.reshape(1, 32), W4p, 32, 16)
    acc4 = _make_scatter(16)(h4s, srcp_g, dst_g)
    b4p = jnp.pad(b4, (0, 15)).reshape(1, 16)
    x4 = _tc_layer_last(acc4, h4s, dinv, b4p, 16)

    xc = jnp.concatenate(
        [x1, x2, x3, x4[:, :1], jnp.zeros((NP, 31), jnp.float32)], axis=1)
    keys = x4[:, 0]

    sel = _sc_sortpool_kernel()(keys, bounds, xc)

    W5p = jnp.pad(conv5_w[:, 0, :].T, ((0, 31), (0, 0)))
    c5 = _tc_head1(sel, W5p, conv5_b.reshape(1, 16))
    hp = c5.reshape(NG, 32, 16)[:, :K, :].reshape(NG, K * 16)

    W6 = conv6_w.transpose(2, 1, 0).reshape(80, 32)
    fc1p = fc1_w.reshape(32, 11, 128).transpose(1, 0, 2).reshape(352, 128)
    return _tc_head2(hp, W6, conv6_b.reshape(1, 32), fc1p,
                     fc1_b.reshape(1, 128), fc2_w, fc2_b.reshape(1, 10))


# speedup vs baseline: 1.9447x; 1.0117x over previous
import functools

import jax
import jax.numpy as jnp
from jax import lax
from jax.experimental import pallas as pl
from jax.experimental.pallas import tpu as pltpu
from jax.experimental.pallas import tpu_sc as plsc

N = 10000
NP = 10240
E = 320000
F0 = 128
NG = 64
K = 30
NC = 2
NS = 16
NW = NC * NS
CPW = 80
EPT = CPW * 128
EPAD = NW * EPT
RPT = NP // NS
GRP = 8
DUMMY = N

def _mesh():
    return plsc.VectorSubcoreMesh(core_axis_name="c", subcore_axis_name="s",
                                  num_cores=NC, num_subcores=NS)


@functools.cache
def _sc_prep_kernel():
    return functools.partial(
        pl.kernel,
        mesh=_mesh(),
        compiler_params=pltpu.CompilerParams(needs_layout_passes=False),
        out_type=[
            jax.ShapeDtypeStruct((NC, NP), jnp.float32),
            jax.ShapeDtypeStruct((NW, CPW, 128), jnp.int32),
            jax.ShapeDtypeStruct((128,), jnp.int32),
        ],
        scratch_types=[
            pltpu.VMEM((CPW, 128), jnp.int32),
            pltpu.VMEM((CPW, 128), jnp.int32),
            pltpu.VMEM((CPW, 128), jnp.int32),
            pltpu.VMEM((NP,), jnp.float32),
            pltpu.VMEM_SHARED((NS, NP), jnp.float32),
            pltpu.VMEM((NS, RPT), jnp.float32),
            pltpu.VMEM((RPT,), jnp.float32),
            pltpu.VMEM((N,), jnp.int32),
            pltpu.VMEM((64,), jnp.int32),
            pltpu.VMEM((128,), jnp.int32),
        ],
    )(_sc_prep_body)


def _sc_prep_body(src_hbm, dst_hbm, batch_hbm, deg_out, srcp_out, bounds_out,
             src_v, dst_v, srcp_v, deg_v, shared_deg, part_v, red_v,
             batch_v, counts_v, bounds_v):
    cid = lax.axis_index("c")
    sid = lax.axis_index("s")
    wid = sid * NC + cid

    zf = jnp.zeros((16,), jnp.float32)

    def _zero(i, _):
        deg_v[pl.ds(i * 16, 16)] = zf
        return 0
    lax.fori_loop(0, NP // 16, _zero, 0)

    pltpu.sync_copy(src_hbm.at[wid], src_v)
    pltpu.sync_copy(dst_hbm.at[wid], dst_v)

    dummy16 = jnp.full((16,), DUMMY, jnp.int32)

    def _edges(j, _):
        for k in range(8):
            s = src_v[j, pl.ds(k * 16, 16)]
            d = dst_v[j, pl.ds(k * 16, 16)]
            m = s != d
            plsc.addupdate_scatter(
                deg_v, [d], jnp.where(m, 1.0, 0.0).astype(jnp.float32))
            srcp_v[j, pl.ds(k * 16, 16)] = jnp.where(m, s, dummy16)
        return 0
    lax.fori_loop(0, CPW, _edges, 0)
    pltpu.sync_copy(srcp_v, srcp_out.at[wid])

    pltpu.sync_copy(deg_v, shared_deg.at[sid])
    plsc.subcore_barrier()
    pltpu.sync_copy(shared_deg.at[:, pl.ds(sid * RPT, RPT)], part_v)

    def _red(i, _):
        acc = jnp.zeros((16,), jnp.float32)
        for k in range(NS):
            acc = acc + part_v[k, pl.ds(i * 16, 16)]
        red_v[pl.ds(i * 16, 16)] = acc
        return 0
    lax.fori_loop(0, RPT // 16, _red, 0)
    pltpu.sync_copy(red_v, deg_out.at[cid, pl.ds(sid * RPT, RPT)])

    @pl.when(jnp.logical_and(cid == 0, sid == 0))
    def _bounds():
        pltpu.sync_copy(batch_hbm, batch_v)
        zi = jnp.zeros((16,), jnp.int32)
        for i in range(4):
            counts_v[pl.ds(i * 16, 16)] = zi
        ones_i = jnp.ones((16,), jnp.int32)

        def _cnt(i, _):
            b = batch_v[pl.ds(i * 16, 16)]
            plsc.addupdate_scatter(counts_v, [b], ones_i)
            return 0
        lax.fori_loop(0, N // 16, _cnt, 0)

        carry = jnp.int32(0)
        for g in range(4):
            c = counts_v[pl.ds(g * 16, 16)]
            cs = plsc.cumsum(c)
            bounds_v[pl.ds(g * 16, 16)] = carry + cs - c
            bounds_v[pl.ds(64 + g * 16, 16)] = carry + cs
            carry = carry + jnp.sum(c)
        pltpu.sync_copy(bounds_v, bounds_out)


@functools.cache
def _make_scatter(F):
    @functools.partial(
        pl.kernel,
        mesh=_mesh(),
        compiler_params=pltpu.CompilerParams(needs_layout_passes=False,
                                             use_tc_tiling_on_sc=False),
        out_type=jax.ShapeDtypeStruct((NC, NP, F), jnp.float32),
        scratch_types=[
            pltpu.VMEM((CPW // GRP, GRP * 128), jnp.int32),
            pltpu.VMEM((CPW // GRP, GRP * 128), jnp.int32),
            pltpu.VMEM((GRP * 128, F), jnp.float32),
            pltpu.VMEM((RPT, F), jnp.float32),
            pltpu.VMEM_SHARED((NP, F), jnp.float32),
            pltpu.VMEM_SHARED((NP, F), jnp.float32),
            pltpu.SemaphoreType.DMA,
        ],
    )
    def _scatter(hs_hbm, srcp_hbm, dst_hbm, acc_out,
                 sidx, didx, rows, zbuf, acc_sh, hs_sh, sem):
        cid = lax.axis_index("c")
        sid = lax.axis_index("s")
        wid = sid * NC + cid

        zf = jnp.zeros((16,), jnp.float32)

        def _zero(i, _):
            for k in range(F // 16):
                zbuf[i, pl.ds(k * 16, 16)] = zf
            return 0
        lax.fori_loop(0, RPT, _zero, 0)
        pltpu.sync_copy(zbuf, acc_sh.at[pl.ds(sid * RPT, RPT), :])
        pltpu.sync_copy(hs_hbm.at[pl.ds(sid * RPT, RPT), :],
                        hs_sh.at[pl.ds(sid * RPT, RPT), :])

        pltpu.sync_copy(srcp_hbm.at[wid], sidx)
        pltpu.sync_copy(dst_hbm.at[wid], didx)
        plsc.subcore_barrier()

        def _edge_chunk(j, _):
            pltpu.async_copy(hs_sh.at[sidx.at[j]], rows, sem).wait()
            pltpu.sync_copy(rows, acc_sh.at[didx.at[j]], add=True)
            return 0
        lax.fori_loop(0, CPW // GRP, _edge_chunk, 0)

        plsc.subcore_barrier()
        pltpu.sync_copy(acc_sh.at[pl.ds(sid * RPT, RPT), :], zbuf)
        pltpu.sync_copy(zbuf, acc_out.at[cid, pl.ds(sid * RPT, RPT), :])

    return _scatter


GPT = NG // NW
_NEG_INF = float("-inf")
_IMAX = 2147483647


@functools.cache
def _sc_sortpool_kernel():
    return functools.partial(
        pl.kernel,
        mesh=_mesh(),
        compiler_params=pltpu.CompilerParams(needs_layout_passes=False),
        out_type=jax.ShapeDtypeStruct((NG * 32, 128), jnp.float32),
        scratch_types=[
            pltpu.VMEM((NP,), jnp.float32),
            pltpu.VMEM((160,), jnp.int32),
            pltpu.VMEM((32,), jnp.int32),
            pltpu.VMEM((32, 128), jnp.float32),
            pltpu.SemaphoreType.DMA,
        ],
    )(_sc_sortpool_body)


def _sc_sortpool_body(keys_hbm, bounds_hbm, xc_hbm, sel_out,
                 keys_v, bounds_v, idx_buf, rows, sem):
    cid = lax.axis_index("c")
    sid = lax.axis_index("s")
    wid = sid * NC + cid
    pltpu.sync_copy(keys_hbm, keys_v)
    pltpu.sync_copy(bounds_hbm, bounds_v.at[pl.ds(0, 128)])

    lane = jnp.arange(16, dtype=jnp.int32)
    neg16 = jnp.full((16,), _NEG_INF, jnp.float32)
    dummy16 = jnp.full((16,), DUMMY, jnp.int32)

    for gi in range(GPT):
        g = wid * GPT + gi
        start = bounds_v[pl.ds(g, 16)][0]
        end = bounds_v[pl.ds(64 + g, 16)][0]
        c_lo = start // 16
        c_hi = (end + 15) // 16

        res = [dummy16, dummy16]
        for r in range(K):
            def _scan(c, carry):
                m_v, i_v = carry
                base = c * 16
                kv = keys_v[pl.ds(base, 16)]
                gidx = base + lane
                valid = jnp.logical_and(gidx >= start, gidx < end)
                kv = jnp.where(valid, kv, neg16)
                upd = kv > m_v
                return jnp.where(upd, kv, m_v), jnp.where(upd, gidx, i_v)

            m_v, i_v = lax.fori_loop(
                c_lo, c_hi, _scan,
                (neg16, jnp.zeros((16,), jnp.int32)))
            m = jnp.max(m_v)
            idx = jnp.min(jnp.where(m_v == m, i_v, _IMAX))
            is_valid = m > _NEG_INF
            idx_final = jnp.where(is_valid, idx, DUMMY)
            plsc.store_scatter(
                keys_v, [jnp.full((16,), idx, jnp.int32)], neg16,
                mask=jnp.logical_and(lane == 0, is_valid))
            q, sl = divmod(r, 16)
            res[q] = jnp.where(lane == sl, idx_final, res[q])

        idx_buf[pl.ds(0, 16)] = res[0]
        idx_buf[pl.ds(16, 16)] = res[1]
        pltpu.async_copy(xc_hbm.at[idx_buf], rows, sem).wait()
        pltpu.sync_copy(rows, sel_out.at[pl.ds(g * 32, 32), :])


_BR = 1024


def _tc_prep(deg3, x_pad, W1):
    def body(deg_ref, x_ref, w_ref, dinv_ref, hs_ref):
        deg = deg_ref[0] + deg_ref[1]
        dinv = jnp.where(deg > 0, lax.rsqrt(deg), 0.0)
        dinv_ref[...] = dinv
        h = jnp.dot(x_ref[...], w_ref[...],
                    preferred_element_type=jnp.float32)
        hs_ref[...] = dinv * h

    return pl.pallas_call(
        body,
        grid=(NP // _BR,),
        in_specs=[
            pl.BlockSpec((NC, _BR, 1), lambda i: (0, i, 0)),
            pl.BlockSpec((_BR, F0), lambda i: (i, 0)),
            pl.BlockSpec((F0, 32), lambda i: (0, 0)),
        ],
        out_specs=[
            pl.BlockSpec((_BR, 1), lambda i: (i, 0)),
            pl.BlockSpec((_BR, 32), lambda i: (i, 0)),
        ],
        out_shape=[
            jax.ShapeDtypeStruct((NP, 1), jnp.float32),
            jax.ShapeDtypeStruct((NP, 32), jnp.float32),
        ],
    )(deg3, x_pad, W1)


def _tc_layer(acc, hs, dinv, b, Wn, Fin, Fn):
    def body(acc_ref, hs_ref, dinv_ref, b_ref, wn_ref, x_ref, hn_ref):
        a = acc_ref[0] + acc_ref[1] + hs_ref[...]
        dinv = dinv_ref[...]
        xv = jnp.tanh(dinv * a + b_ref[...])
        xv = jnp.where(dinv > 0, xv, 0.0)
        x_ref[...] = xv
        hn_ref[...] = dinv * jnp.dot(xv, wn_ref[...],
                                     preferred_element_type=jnp.float32)

    return pl.pallas_call(
        body,
        grid=(NP // _BR,),
        in_specs=[
            pl.BlockSpec((NC, _BR, Fin), lambda i: (0, i, 0)),
            pl.BlockSpec((_BR, Fin), lambda i: (i, 0)),
            pl.BlockSpec((_BR, 1), lambda i: (i, 0)),
            pl.BlockSpec((1, Fin), lambda i: (0, 0)),
            pl.BlockSpec((Fin, Fn), lambda i: (0, 0)),
        ],
        out_specs=[
            pl.BlockSpec((_BR, Fin), lambda i: (i, 0)),
            pl.BlockSpec((_BR, Fn), lambda i: (i, 0)),
        ],
        out_shape=[
            jax.ShapeDtypeStruct((NP, Fin), jnp.float32),
            jax.ShapeDtypeStruct((NP, Fn), jnp.float32),
        ],
    )(acc, hs, dinv, b, Wn)


def _tc_layer_last(acc, hs, dinv, b, Fin):
    def body(acc_ref, hs_ref, dinv_ref, b_ref, x_ref):
        a = acc_ref[0] + acc_ref[1] + hs_ref[...]
        dinv = dinv_ref[...]
        xv = jnp.tanh(dinv * a + b_ref[...])
        x_ref[...] = jnp.where(dinv > 0, xv, 0.0)

    return pl.pallas_call(
        body,
        grid=(NP // _BR,),
        in_specs=[
            pl.BlockSpec((NC, _BR, Fin), lambda i: (0, i, 0)),
            pl.BlockSpec((_BR, Fin), lambda i: (i, 0)),
            pl.BlockSpec((_BR, 1), lambda i: (i, 0)),
            pl.BlockSpec((1, Fin), lambda i: (0, 0)),
        ],
        out_specs=pl.BlockSpec((_BR, Fin), lambda i: (i, 0)),
        out_shape=jax.ShapeDtypeStruct((NP, Fin), jnp.float32),
    )(acc, hs, dinv, b)


def _tc_head1(sel, W5p, b5):
    def body(s_ref, w_ref, b_ref, o_ref):
        o_ref[...] = jnp.maximum(
            jnp.dot(s_ref[...], w_ref[...],
                    preferred_element_type=jnp.float32) + b_ref[...], 0.0)

    return pl.pallas_call(
        body,
        out_shape=jax.ShapeDtypeStruct((NG * 32, 16), jnp.float32),
    )(sel, W5p, b5)


def _tc_head2(hp, W6, b6, fc1p, fc1b, fc2, fc2b):
    def body(hp_ref, w6_ref, b6_ref, f1_ref, f1b_ref, f2_ref, f2b_ref, o_ref):
        hp = hp_ref[...]
        pooled = jnp.concatenate(
            [jnp.maximum(hp[:, 32 * j:32 * j + 16],
                         hp[:, 32 * j + 16:32 * j + 32]) for j in range(15)],
            axis=1)
        w6 = w6_ref[...]
        b6 = b6_ref[...]
        h6 = jnp.concatenate(
            [jnp.maximum(
                jnp.dot(pooled[:, 16 * l:16 * l + 80], w6,
                        preferred_element_type=jnp.float32) + b6, 0.0)
             for l in range(11)], axis=1)
        h = jnp.maximum(
            jnp.dot(h6, f1_ref[...],
                    preferred_element_type=jnp.float32) + f1b_ref[...], 0.0)
        logits = jnp.dot(h, f2_ref[...],
                         preferred_element_type=jnp.float32) + f2b_ref[...]
        m = jnp.max(logits, axis=1, keepdims=True)
        lse = jnp.log(jnp.sum(jnp.exp(logits - m), axis=1, keepdims=True)) + m
        o_ref[...] = logits - lse

    return pl.pallas_call(
        body,
        out_shape=jax.ShapeDtypeStruct((NG, 10), jnp.float32),
    )(hp, W6, b6, fc1p, fc1b, fc2, fc2b)


def kernel(x, edge_index, batch, W1, b1, W2, b2, W3, b3, W4, b4,
           conv5_w, conv5_b, conv6_w, conv6_b, fc1_w, fc1_b, fc2_w, fc2_b):
    src = jnp.pad(edge_index[0], (0, EPAD - E)).reshape(NW, CPW, 128)
    dst = jnp.pad(edge_index[1], (0, EPAD - E)).reshape(NW, CPW, 128)
    x_pad = jnp.pad(x, ((0, NP - N), (0, 0)))

    deg_part, srcp, bounds = _sc_prep_kernel()(src, dst, batch)
    dinv, h1s = _tc_prep(deg_part.reshape(NC, NP, 1), x_pad, W1)

    srcp_g = srcp.reshape(NW, CPW // GRP, GRP * 128)
    dst_g = dst.reshape(NW, CPW // GRP, GRP * 128)
    _scatter32 = _make_scatter(32)
    acc1 = _scatter32(h1s, srcp_g, dst_g)
    x1, h2s = _tc_layer(acc1, h1s, dinv, b1.reshape(1, 32), W2, 32, 32)
    acc2 = _scatter32(h2s, srcp_g, dst_g)
    x2, h3s = _tc_layer(acc2, h2s, dinv, b2.reshape(1, 32), W3, 32, 32)
    acc3 = _scatter32(h3s, srcp_g, dst_g)
    W4p = jnp.pad(W4, ((0, 0), (0, 15)))
    x3, h4s = _tc_layer(acc3, h3s, dinv, b3.reshape(1, 32), W4p, 32, 16)
    acc4 = _make_scatter(16)(h4s, srcp_g, dst_g)
    b4p = jnp.pad(b4, (0, 15)).reshape(1, 16)
    x4 = _tc_layer_last(acc4, h4s, dinv, b4p, 16)

    xc = jnp.concatenate(
        [x1, x2, x3, x4[:, :1], jnp.zeros((NP, 31), jnp.float32)], axis=1)
    keys = x4[:, 0]

    sel = _sc_sortpool_kernel()(keys, bounds, xc)

    W5p = jnp.pad(conv5_w[:, 0, :].T, ((0, 31), (0, 0)))
    c5 = _tc_head1(sel, W5p, conv5_b.reshape(1, 16))
    hp = c5.reshape(NG, 32, 16)[:, :K, :].reshape(NG, K * 16)

    W6 = conv6_w.transpose(2, 1, 0).reshape(80, 32)
    fc1p = fc1_w.reshape(32, 11, 128).transpose(1, 0, 2).reshape(352, 128)
    return _tc_head2(hp, W6, conv6_b.reshape(1, 32), fc1p,
                     fc1_b.reshape(1, 128), fc2_w, fc2_b.reshape(1, 10))


# speedup vs baseline: 1.9632x; 1.0095x over previous
import functools

import jax
import jax.numpy as jnp
from jax import lax
from jax.experimental import pallas as pl
from jax.experimental.pallas import tpu as pltpu
from jax.experimental.pallas import tpu_sc as plsc

N = 10000
NP = 10240
E = 320000
F0 = 128
NG = 64
K = 30
NC = 2
NS = 16
NW = NC * NS
CPW = 80
EPT = CPW * 128
EPAD = NW * EPT
RPT = NP // NS
GRP = 8
DUMMY = N

def _mesh():
    return plsc.VectorSubcoreMesh(core_axis_name="c", subcore_axis_name="s",
                                  num_cores=NC, num_subcores=NS)


@functools.cache
def _sc_prep_kernel():
    return functools.partial(
        pl.kernel,
        mesh=_mesh(),
        compiler_params=pltpu.CompilerParams(needs_layout_passes=False),
        out_type=[
            jax.ShapeDtypeStruct((NC, NP), jnp.float32),
            jax.ShapeDtypeStruct((NW, CPW, 128), jnp.int32),
            jax.ShapeDtypeStruct((128,), jnp.int32),
        ],
        scratch_types=[
            pltpu.VMEM((CPW, 128), jnp.int32),
            pltpu.VMEM((CPW, 128), jnp.int32),
            pltpu.VMEM((CPW, 128), jnp.int32),
            pltpu.VMEM((NP,), jnp.float32),
            pltpu.VMEM_SHARED((NS, NP), jnp.float32),
            pltpu.VMEM((NS, RPT), jnp.float32),
            pltpu.VMEM((RPT,), jnp.float32),
            pltpu.VMEM((N,), jnp.int32),
            pltpu.VMEM((64,), jnp.int32),
            pltpu.VMEM((128,), jnp.int32),
        ],
    )(_sc_prep_body)


def _sc_prep_body(src_hbm, dst_hbm, batch_hbm, deg_out, srcp_out, bounds_out,
             src_v, dst_v, srcp_v, deg_v, shared_deg, part_v, red_v,
             batch_v, counts_v, bounds_v):
    cid = lax.axis_index("c")
    sid = lax.axis_index("s")
    wid = sid * NC + cid

    zf = jnp.zeros((16,), jnp.float32)

    def _zero(i, _):
        deg_v[pl.ds(i * 16, 16)] = zf
        return 0
    lax.fori_loop(0, NP // 16, _zero, 0)

    pltpu.sync_copy(src_hbm.at[wid], src_v)
    pltpu.sync_copy(dst_hbm.at[wid], dst_v)

    dummy16 = jnp.full((16,), DUMMY, jnp.int32)

    def _edges(j, _):
        for k in range(8):
            s = src_v[j, pl.ds(k * 16, 16)]
            d = dst_v[j, pl.ds(k * 16, 16)]
            m = s != d
            plsc.addupdate_scatter(
                deg_v, [d], jnp.where(m, 1.0, 0.0).astype(jnp.float32))
            srcp_v[j, pl.ds(k * 16, 16)] = jnp.where(m, s, dummy16)
        return 0
    lax.fori_loop(0, CPW, _edges, 0)
    pltpu.sync_copy(srcp_v, srcp_out.at[wid])

    pltpu.sync_copy(deg_v, shared_deg.at[sid])
    plsc.subcore_barrier()
    pltpu.sync_copy(shared_deg.at[:, pl.ds(sid * RPT, RPT)], part_v)

    def _red(i, _):
        acc = jnp.zeros((16,), jnp.float32)
        for k in range(NS):
            acc = acc + part_v[k, pl.ds(i * 16, 16)]
        red_v[pl.ds(i * 16, 16)] = acc
        return 0
    lax.fori_loop(0, RPT // 16, _red, 0)
    pltpu.sync_copy(red_v, deg_out.at[cid, pl.ds(sid * RPT, RPT)])

    @pl.when(jnp.logical_and(cid == 0, sid == 0))
    def _bounds():
        pltpu.sync_copy(batch_hbm, batch_v)
        zi = jnp.zeros((16,), jnp.int32)
        for i in range(4):
            counts_v[pl.ds(i * 16, 16)] = zi
        ones_i = jnp.ones((16,), jnp.int32)

        def _cnt(i, _):
            b = batch_v[pl.ds(i * 16, 16)]
            plsc.addupdate_scatter(counts_v, [b], ones_i)
            return 0
        lax.fori_loop(0, N // 16, _cnt, 0)

        carry = jnp.int32(0)
        for g in range(4):
            c = counts_v[pl.ds(g * 16, 16)]
            cs = plsc.cumsum(c)
            bounds_v[pl.ds(g * 16, 16)] = carry + cs - c
            bounds_v[pl.ds(64 + g * 16, 16)] = carry + cs
            carry = carry + jnp.sum(c)
        pltpu.sync_copy(bounds_v, bounds_out)


@functools.cache
def _make_scatter(F):
    @functools.partial(
        pl.kernel,
        mesh=_mesh(),
        compiler_params=pltpu.CompilerParams(needs_layout_passes=False,
                                             use_tc_tiling_on_sc=False),
        out_type=jax.ShapeDtypeStruct((NC, NP, F), jnp.float32),
        scratch_types=[
            pltpu.VMEM((CPW // GRP, GRP * 128), jnp.int32),
            pltpu.VMEM((CPW // GRP, GRP * 128), jnp.int32),
            pltpu.VMEM((GRP * 128, F), jnp.float32),
            pltpu.VMEM((RPT, F), jnp.float32),
            pltpu.VMEM_SHARED((NP, F), jnp.float32),
            pltpu.VMEM_SHARED((NP, F), jnp.float32),
            pltpu.SemaphoreType.DMA,
        ],
    )
    def _scatter(hs_hbm, srcp_hbm, dst_hbm, acc_out,
                 sidx, didx, rows, zbuf, acc_sh, hs_sh, sem):
        cid = lax.axis_index("c")
        sid = lax.axis_index("s")
        wid = sid * NC + cid

        zf = jnp.zeros((16,), jnp.float32)

        def _zero(i, _):
            for k in range(F // 16):
                zbuf[i, pl.ds(k * 16, 16)] = zf
            return 0
        lax.fori_loop(0, RPT, _zero, 0)
        pltpu.sync_copy(zbuf, acc_sh.at[pl.ds(sid * RPT, RPT), :])
        pltpu.sync_copy(hs_hbm.at[pl.ds(sid * RPT, RPT), :],
                        hs_sh.at[pl.ds(sid * RPT, RPT), :])

        pltpu.sync_copy(srcp_hbm.at[wid], sidx)
        pltpu.sync_copy(dst_hbm.at[wid], didx)
        plsc.subcore_barrier()

        def _edge_chunk(j, _):
            pltpu.async_copy(hs_sh.at[sidx.at[j]], rows, sem).wait()
            pltpu.sync_copy(rows, acc_sh.at[didx.at[j]], add=True)
            return 0
        lax.fori_loop(0, CPW // GRP, _edge_chunk, 0)

        plsc.subcore_barrier()
        pltpu.sync_copy(acc_sh.at[pl.ds(sid * RPT, RPT), :], zbuf)
        pltpu.sync_copy(zbuf, acc_out.at[cid, pl.ds(sid * RPT, RPT), :])

    return _scatter


GPT = NG // NW
_NEG_INF = float("-inf")
_IMAX = 2147483647


@functools.cache
def _sc_sortpool_kernel():
    return functools.partial(
        pl.kernel,
        mesh=_mesh(),
        compiler_params=pltpu.CompilerParams(needs_layout_passes=False),
        out_type=jax.ShapeDtypeStruct((NG * 32, 128), jnp.float32),
        scratch_types=[
            pltpu.VMEM((NP,), jnp.float32),
            pltpu.VMEM((160,), jnp.int32),
            pltpu.VMEM((32,), jnp.int32),
            pltpu.VMEM((32, 128), jnp.float32),
            pltpu.SemaphoreType.DMA,
        ],
    )(_sc_sortpool_body)


def _sc_sortpool_body(keys_hbm, bounds_hbm, xc_hbm, sel_out,
                 keys_v, bounds_v, idx_buf, rows, sem):
    cid = lax.axis_index("c")
    sid = lax.axis_index("s")
    wid = sid * NC + cid
    pltpu.sync_copy(keys_hbm, keys_v)
    pltpu.sync_copy(bounds_hbm, bounds_v.at[pl.ds(0, 128)])

    lane = jnp.arange(16, dtype=jnp.int32)
    neg16 = jnp.full((16,), _NEG_INF, jnp.float32)
    dummy16 = jnp.full((16,), DUMMY, jnp.int32)

    for gi in range(GPT):
        g = wid * GPT + gi
        start = bounds_v[pl.ds(g, 16)][0]
        end = bounds_v[pl.ds(64 + g, 16)][0]
        c_lo = start // 16
        c_hi = (end + 15) // 16

        res = [dummy16, dummy16]
        for r in range(K):
            def _scan(c, carry):
                m_v, i_v = carry
                base = c * 16
                kv = keys_v[pl.ds(base, 16)]
                gidx = base + lane
                valid = jnp.logical_and(gidx >= start, gidx < end)
                kv = jnp.where(valid, kv, neg16)
                upd = kv > m_v
                return jnp.where(upd, kv, m_v), jnp.where(upd, gidx, i_v)

            m_v, i_v = lax.fori_loop(
                c_lo, c_hi, _scan,
                (neg16, jnp.zeros((16,), jnp.int32)))
            m = jnp.max(m_v)
            idx = jnp.min(jnp.where(m_v == m, i_v, _IMAX))
            is_valid = m > _NEG_INF
            idx_final = jnp.where(is_valid, idx, DUMMY)
            plsc.store_scatter(
                keys_v, [jnp.full((16,), idx, jnp.int32)], neg16,
                mask=jnp.logical_and(lane == 0, is_valid))
            q, sl = divmod(r, 16)
            res[q] = jnp.where(lane == sl, idx_final, res[q])

        idx_buf[pl.ds(0, 16)] = res[0]
        idx_buf[pl.ds(16, 16)] = res[1]
        pltpu.async_copy(xc_hbm.at[idx_buf], rows, sem).wait()
        pltpu.sync_copy(rows, sel_out.at[pl.ds(g * 32, 32), :])


_BR = 1024


def _tc_prep(deg3, x_pad, W1):
    def body(deg_ref, x_ref, w_ref, dinv_ref, hs_ref):
        deg = deg_ref[0] + deg_ref[1]
        dinv = jnp.where(deg > 0, lax.rsqrt(deg), 0.0)
        dinv_ref[...] = dinv
        h = jnp.dot(x_ref[...], w_ref[...],
                    preferred_element_type=jnp.float32)
        hs_ref[...] = dinv * h

    return pl.pallas_call(
        body,
        grid=(NP // _BR,),
        in_specs=[
            pl.BlockSpec((NC, _BR, 1), lambda i: (0, i, 0)),
            pl.BlockSpec((_BR, F0), lambda i: (i, 0)),
            pl.BlockSpec((F0, 32), lambda i: (0, 0)),
        ],
        out_specs=[
            pl.BlockSpec((_BR, 1), lambda i: (i, 0)),
            pl.BlockSpec((_BR, 32), lambda i: (i, 0)),
        ],
        out_shape=[
            jax.ShapeDtypeStruct((NP, 1), jnp.float32),
            jax.ShapeDtypeStruct((NP, 32), jnp.float32),
        ],
    )(deg3, x_pad, W1)


def _tc_layer(acc, hs, dinv, b, Wn, Fin, Fn):
    def body(acc_ref, hs_ref, dinv_ref, b_ref, wn_ref, x_ref, hn_ref):
        a = acc_ref[0] + acc_ref[1] + hs_ref[...]
        dinv = dinv_ref[...]
        xv = jnp.tanh(dinv * a + b_ref[...])
        xv = jnp.where(dinv > 0, xv, 0.0)
        x_ref[...] = xv
        hn_ref[...] = dinv * jnp.dot(xv, wn_ref[...],
                                     preferred_element_type=jnp.float32)

    return pl.pallas_call(
        body,
        grid=(NP // _BR,),
        in_specs=[
            pl.BlockSpec((NC, _BR, Fin), lambda i: (0, i, 0)),
            pl.BlockSpec((_BR, Fin), lambda i: (i, 0)),
            pl.BlockSpec((_BR, 1), lambda i: (i, 0)),
            pl.BlockSpec((1, Fin), lambda i: (0, 0)),
            pl.BlockSpec((Fin, Fn), lambda i: (0, 0)),
        ],
        out_specs=[
            pl.BlockSpec((_BR, Fin), lambda i: (i, 0)),
            pl.BlockSpec((_BR, Fn), lambda i: (i, 0)),
        ],
        out_shape=[
            jax.ShapeDtypeStruct((NP, Fin), jnp.float32),
            jax.ShapeDtypeStruct((NP, Fn), jnp.float32),
        ],
    )(acc, hs, dinv, b, Wn)


def _tc_layer_last(acc, hs, dinv, b, Fin):
    def body(acc_ref, hs_ref, dinv_ref, b_ref, x_ref):
        a = acc_ref[0] + acc_ref[1] + hs_ref[...]
        dinv = dinv_ref[...]
        xv = jnp.tanh(dinv * a + b_ref[...])
        x_ref[...] = jnp.where(dinv > 0, xv, 0.0)

    return pl.pallas_call(
        body,
        grid=(NP // _BR,),
        in_specs=[
            pl.BlockSpec((NC, _BR, Fin), lambda i: (0, i, 0)),
            pl.BlockSpec((_BR, Fin), lambda i: (i, 0)),
            pl.BlockSpec((_BR, 1), lambda i: (i, 0)),
            pl.BlockSpec((1, Fin), lambda i: (0, 0)),
        ],
        out_specs=pl.BlockSpec((_BR, Fin), lambda i: (i, 0)),
        out_shape=jax.ShapeDtypeStruct((NP, Fin), jnp.float32),
    )(acc, hs, dinv, b)


def _tc_head(sel, W5p, b5, W6, b6, fc1p, fc1b, fc2, fc2b):
    def body(s_ref, w5_ref, b5_ref, w6_ref, b6_ref, f1_ref, f1b_ref,
             f2_ref, f2b_ref, o_ref):
        c5 = jnp.maximum(
            jnp.dot(s_ref[...], w5_ref[...],
                    preferred_element_type=jnp.float32) + b5_ref[...], 0.0)
        c4 = c5.reshape(NG, 16, 2, 16)
        p = jnp.max(c4, axis=2)
        w6 = w6_ref[...]
        b6 = b6_ref[...]
        outs = []
        for l in range(11):
            a = jnp.zeros((NG, 32), jnp.float32)
            for t in range(5):
                a = a + jnp.dot(p[:, l + t, :], w6[16 * t:16 * t + 16, :],
                                preferred_element_type=jnp.float32)
            outs.append(jnp.maximum(a + b6, 0.0))
        h6 = jnp.concatenate(outs, axis=1)
        h = jnp.maximum(
            jnp.dot(h6, f1_ref[...],
                    preferred_element_type=jnp.float32) + f1b_ref[...], 0.0)
        logits = jnp.dot(h, f2_ref[...],
                         preferred_element_type=jnp.float32) + f2b_ref[...]
        m = jnp.max(logits, axis=1, keepdims=True)
        lse = jnp.log(jnp.sum(jnp.exp(logits - m), axis=1, keepdims=True)) + m
        o_ref[...] = logits - lse

    return pl.pallas_call(
        body,
        out_shape=jax.ShapeDtypeStruct((NG, 10), jnp.float32),
    )(sel, W5p, b5, W6, b6, fc1p, fc1b, fc2, fc2b)


def kernel(x, edge_index, batch, W1, b1, W2, b2, W3, b3, W4, b4,
           conv5_w, conv5_b, conv6_w, conv6_b, fc1_w, fc1_b, fc2_w, fc2_b):
    src = jnp.pad(edge_index[0], (0, EPAD - E)).reshape(NW, CPW, 128)
    dst = jnp.pad(edge_index[1], (0, EPAD - E)).reshape(NW, CPW, 128)
    x_pad = jnp.pad(x, ((0, NP - N), (0, 0)))

    deg_part, srcp, bounds = _sc_prep_kernel()(src, dst, batch)
    dinv, h1s = _tc_prep(deg_part.reshape(NC, NP, 1), x_pad, W1)

    srcp_g = srcp.reshape(NW, CPW // GRP, GRP * 128)
    dst_g = dst.reshape(NW, CPW // GRP, GRP * 128)
    _scatter32 = _make_scatter(32)
    acc1 = _scatter32(h1s, srcp_g, dst_g)
    x1, h2s = _tc_layer(acc1, h1s, dinv, b1.reshape(1, 32), W2, 32, 32)
    acc2 = _scatter32(h2s, srcp_g, dst_g)
    x2, h3s = _tc_layer(acc2, h2s, dinv, b2.reshape(1, 32), W3, 32, 32)
    acc3 = _scatter32(h3s, srcp_g, dst_g)
    W4p = jnp.pad(W4, ((0, 0), (0, 15)))
    x3, h4s = _tc_layer(acc3, h3s, dinv, b3.reshape(1, 32), W4p, 32, 16)
    acc4 = _make_scatter(16)(h4s, srcp_g, dst_g)
    b4p = jnp.pad(b4, (0, 15)).reshape(1, 16)
    x4 = _tc_layer_last(acc4, h4s, dinv, b4p, 16)

    xc = jnp.concatenate(
        [x1, x2, x3, x4[:, :1], jnp.zeros((NP, 31), jnp.float32)], axis=1)
    keys = x4[:, 0]

    sel = _sc_sortpool_kernel()(keys, bounds, xc)

    W5p = jnp.pad(conv5_w[:, 0, :].T, ((0, 31), (0, 0)))
    W6 = conv6_w.transpose(2, 1, 0).reshape(80, 32)
    fc1p = fc1_w.reshape(32, 11, 128).transpose(1, 0, 2).reshape(352, 128)
    return _tc_head(sel, W5p, conv5_b.reshape(1, 16), W6,
                    conv6_b.reshape(1, 32), fc1p, fc1_b.reshape(1, 128),
                    fc2_w, fc2_b.reshape(1, 10))


# speedup vs baseline: 2.0082x; 1.0229x over previous
import functools

import jax
import jax.numpy as jnp
from jax import lax
from jax.experimental import pallas as pl
from jax.experimental.pallas import tpu as pltpu
from jax.experimental.pallas import tpu_sc as plsc

N = 10000
NP = 10240
E = 320000
F0 = 128
NG = 64
K = 30
NC = 2
NS = 16
NW = NC * NS
CPW = 80
EPT = CPW * 128
EPAD = NW * EPT
RPT = NP // NS
GRP = 8
DUMMY = N

def _mesh():
    return plsc.VectorSubcoreMesh(core_axis_name="c", subcore_axis_name="s",
                                  num_cores=NC, num_subcores=NS)


@functools.cache
def _sc_prep_kernel():
    return functools.partial(
        pl.kernel,
        mesh=_mesh(),
        compiler_params=pltpu.CompilerParams(needs_layout_passes=False),
        out_type=[
            jax.ShapeDtypeStruct((NC, NP), jnp.float32),
            jax.ShapeDtypeStruct((NW, CPW, 128), jnp.int32),
            jax.ShapeDtypeStruct((128,), jnp.int32),
        ],
        scratch_types=[
            pltpu.VMEM((CPW, 128), jnp.int32),
            pltpu.VMEM((CPW, 128), jnp.int32),
            pltpu.VMEM((CPW, 128), jnp.int32),
            pltpu.VMEM((NP,), jnp.float32),
            pltpu.VMEM_SHARED((NS, NP), jnp.float32),
            pltpu.VMEM((NS, RPT), jnp.float32),
            pltpu.VMEM((RPT,), jnp.float32),
            pltpu.VMEM((N,), jnp.int32),
            pltpu.VMEM((64,), jnp.int32),
            pltpu.VMEM((128,), jnp.int32),
        ],
    )(_sc_prep_body)


def _sc_prep_body(src_hbm, dst_hbm, batch_hbm, deg_out, srcp_out, bounds_out,
             src_v, dst_v, srcp_v, deg_v, shared_deg, part_v, red_v,
             batch_v, counts_v, bounds_v):
    cid = lax.axis_index("c")
    sid = lax.axis_index("s")
    wid = sid * NC + cid

    zf = jnp.zeros((16,), jnp.float32)

    def _zero(i, _):
        deg_v[pl.ds(i * 16, 16)] = zf
        return 0
    lax.fori_loop(0, NP // 16, _zero, 0)

    pltpu.sync_copy(src_hbm.at[wid], src_v)
    pltpu.sync_copy(dst_hbm.at[wid], dst_v)

    dummy16 = jnp.full((16,), DUMMY, jnp.int32)

    def _edges(j, _):
        for k in range(8):
            s = src_v[j, pl.ds(k * 16, 16)]
            d = dst_v[j, pl.ds(k * 16, 16)]
            m = s != d
            plsc.addupdate_scatter(
                deg_v, [d], jnp.where(m, 1.0, 0.0).astype(jnp.float32))
            srcp_v[j, pl.ds(k * 16, 16)] = jnp.where(m, s, dummy16)
        return 0
    lax.fori_loop(0, CPW, _edges, 0)
    pltpu.sync_copy(srcp_v, srcp_out.at[wid])

    pltpu.sync_copy(deg_v, shared_deg.at[sid])
    plsc.subcore_barrier()
    pltpu.sync_copy(shared_deg.at[:, pl.ds(sid * RPT, RPT)], part_v)

    def _red(i, _):
        acc = jnp.zeros((16,), jnp.float32)
        for k in range(NS):
            acc = acc + part_v[k, pl.ds(i * 16, 16)]
        red_v[pl.ds(i * 16, 16)] = acc
        return 0
    lax.fori_loop(0, RPT // 16, _red, 0)
    pltpu.sync_copy(red_v, deg_out.at[cid, pl.ds(sid * RPT, RPT)])

    @pl.when(jnp.logical_and(cid == 0, sid == 0))
    def _bounds():
        pltpu.sync_copy(batch_hbm, batch_v)
        zi = jnp.zeros((16,), jnp.int32)
        for i in range(4):
            counts_v[pl.ds(i * 16, 16)] = zi
        ones_i = jnp.ones((16,), jnp.int32)

        def _cnt(i, _):
            b = batch_v[pl.ds(i * 16, 16)]
            plsc.addupdate_scatter(counts_v, [b], ones_i)
            return 0
        lax.fori_loop(0, N // 16, _cnt, 0)

        carry = jnp.int32(0)
        for g in range(4):
            c = counts_v[pl.ds(g * 16, 16)]
            cs = plsc.cumsum(c)
            bounds_v[pl.ds(g * 16, 16)] = carry + cs - c
            bounds_v[pl.ds(64 + g * 16, 16)] = carry + cs
            carry = carry + jnp.sum(c)
        pltpu.sync_copy(bounds_v, bounds_out)


@functools.cache
def _make_scatter(F):
    @functools.partial(
        pl.kernel,
        mesh=_mesh(),
        compiler_params=pltpu.CompilerParams(needs_layout_passes=False,
                                             use_tc_tiling_on_sc=False),
        out_type=jax.ShapeDtypeStruct((NC, NP, F), jnp.float32),
        scratch_types=[
            pltpu.VMEM((CPW // GRP, GRP * 128), jnp.int32),
            pltpu.VMEM((CPW // GRP, GRP * 128), jnp.int32),
            pltpu.VMEM((GRP * 128, F), jnp.float32),
            pltpu.VMEM((RPT, F), jnp.float32),
            pltpu.VMEM_SHARED((NP, F), jnp.float32),
            pltpu.VMEM_SHARED((NP, F), jnp.float32),
            pltpu.SemaphoreType.DMA,
        ],
    )
    def _scatter(hs_hbm, srcp_hbm, dst_hbm, acc_out,
                 sidx, didx, rows, zbuf, acc_sh, hs_sh, sem):
        cid = lax.axis_index("c")
        sid = lax.axis_index("s")
        wid = sid * NC + cid

        zf = jnp.zeros((16,), jnp.float32)

        def _zero(i, _):
            for k in range(F // 16):
                zbuf[i, pl.ds(k * 16, 16)] = zf
            return 0
        lax.fori_loop(0, RPT, _zero, 0)
        pltpu.sync_copy(zbuf, acc_sh.at[pl.ds(sid * RPT, RPT), :])
        pltpu.sync_copy(hs_hbm.at[pl.ds(sid * RPT, RPT), :],
                        hs_sh.at[pl.ds(sid * RPT, RPT), :])

        pltpu.sync_copy(srcp_hbm.at[wid], sidx)
        pltpu.sync_copy(dst_hbm.at[wid], didx)
        plsc.subcore_barrier()

        def _edge_chunk(j, _):
            pltpu.async_copy(hs_sh.at[sidx.at[j]], rows, sem).wait()
            pltpu.sync_copy(rows, acc_sh.at[didx.at[j]], add=True)
            return 0
        lax.fori_loop(0, CPW // GRP, _edge_chunk, 0)

        plsc.subcore_barrier()
        pltpu.sync_copy(acc_sh.at[pl.ds(sid * RPT, RPT), :], zbuf)
        pltpu.sync_copy(zbuf, acc_out.at[cid, pl.ds(sid * RPT, RPT), :])

    return _scatter


GPT = NG // NW
_NEG_INF = float("-inf")
_IMAX = 2147483647


@functools.cache
def _sc_sortpool_kernel():
    return functools.partial(
        pl.kernel,
        mesh=_mesh(),
        compiler_params=pltpu.CompilerParams(needs_layout_passes=False),
        out_type=jax.ShapeDtypeStruct((NG * 32, 128), jnp.float32),
        scratch_types=[
            pltpu.VMEM((NP,), jnp.float32),
            pltpu.VMEM((160,), jnp.int32),
            pltpu.VMEM((32,), jnp.int32),
            pltpu.VMEM((32, 128), jnp.float32),
            pltpu.SemaphoreType.DMA,
        ],
    )(_sc_sortpool_body)


def _sc_sortpool_body(keys_hbm, bounds_hbm, xc_hbm, sel_out,
                 keys_v, bounds_v, idx_buf, rows, sem):
    cid = lax.axis_index("c")
    sid = lax.axis_index("s")
    wid = sid * NC + cid
    pltpu.sync_copy(keys_hbm, keys_v)
    pltpu.sync_copy(bounds_hbm, bounds_v.at[pl.ds(0, 128)])

    lane = jnp.arange(16, dtype=jnp.int32)
    neg16 = jnp.full((16,), _NEG_INF, jnp.float32)
    dummy16 = jnp.full((16,), DUMMY, jnp.int32)

    for gi in range(GPT):
        g = wid * GPT + gi
        start = bounds_v[pl.ds(g, 16)][0]
        end = bounds_v[pl.ds(64 + g, 16)][0]
        c_lo = start // 16
        c_hi = (end + 15) // 16

        res = [dummy16, dummy16]
        for r in range(K):
            def _scan(c, carry):
                m_v, i_v = carry
                base = c * 16
                kv = keys_v[pl.ds(base, 16)]
                gidx = base + lane
                valid = jnp.logical_and(gidx >= start, gidx < end)
                kv = jnp.where(valid, kv, neg16)
                upd = kv > m_v
                return jnp.where(upd, kv, m_v), jnp.where(upd, gidx, i_v)

            m_v, i_v = lax.fori_loop(
                c_lo, c_hi, _scan,
                (neg16, jnp.zeros((16,), jnp.int32)))
            m = jnp.max(m_v)
            idx = jnp.min(jnp.where(m_v == m, i_v, _IMAX))
            is_valid = m > _NEG_INF
            idx_final = jnp.where(is_valid, idx, DUMMY)
            plsc.store_scatter(
                keys_v, [jnp.full((16,), idx, jnp.int32)], neg16,
                mask=jnp.logical_and(lane == 0, is_valid))
            q, sl = divmod(r, 16)
            res[q] = jnp.where(lane == sl, idx_final, res[q])

        idx_buf[pl.ds(0, 16)] = res[0]
        idx_buf[pl.ds(16, 16)] = res[1]
        pltpu.async_copy(xc_hbm.at[idx_buf], rows, sem).wait()
        pltpu.sync_copy(rows, sel_out.at[pl.ds(g * 32, 32), :])


_BR = 1024


def _tc_prep(deg3, x_pad, W1):
    def body(deg_ref, x_ref, w_ref, dinv_ref, hs_ref):
        deg = deg_ref[0] + deg_ref[1]
        dinv = jnp.where(deg > 0, lax.rsqrt(deg), 0.0)
        dinv_ref[...] = dinv
        h = jnp.dot(x_ref[...], w_ref[...],
                    preferred_element_type=jnp.float32)
        hs_ref[...] = dinv * h

    return pl.pallas_call(
        body,
        grid=(NP // _BR,),
        in_specs=[
            pl.BlockSpec((NC, _BR, 1), lambda i: (0, i, 0)),
            pl.BlockSpec((_BR, F0), lambda i: (i, 0)),
            pl.BlockSpec((F0, 32), lambda i: (0, 0)),
        ],
        out_specs=[
            pl.BlockSpec((_BR, 1), lambda i: (i, 0)),
            pl.BlockSpec((_BR, 32), lambda i: (i, 0)),
        ],
        out_shape=[
            jax.ShapeDtypeStruct((NP, 1), jnp.float32),
            jax.ShapeDtypeStruct((NP, 32), jnp.float32),
        ],
    )(deg3, x_pad, W1)


def _tc_layer(acc, hs, dinv, b, Wn, Fin, Fn):
    def body(acc_ref, hs_ref, dinv_ref, b_ref, wn_ref, x_ref, hn_ref):
        a = acc_ref[0] + acc_ref[1] + hs_ref[...]
        dinv = dinv_ref[...]
        xv = jnp.tanh(dinv * a + b_ref[...])
        xv = jnp.where(dinv > 0, xv, 0.0)
        x_ref[...] = xv
        hn_ref[...] = dinv * jnp.dot(xv, wn_ref[...],
                                     preferred_element_type=jnp.float32)

    return pl.pallas_call(
        body,
        grid=(NP // _BR,),
        in_specs=[
            pl.BlockSpec((NC, _BR, Fin), lambda i: (0, i, 0)),
            pl.BlockSpec((_BR, Fin), lambda i: (i, 0)),
            pl.BlockSpec((_BR, 1), lambda i: (i, 0)),
            pl.BlockSpec((1, Fin), lambda i: (0, 0)),
            pl.BlockSpec((Fin, Fn), lambda i: (0, 0)),
        ],
        out_specs=[
            pl.BlockSpec((_BR, Fin), lambda i: (i, 0)),
            pl.BlockSpec((_BR, Fn), lambda i: (i, 0)),
        ],
        out_shape=[
            jax.ShapeDtypeStruct((NP, Fin), jnp.float32),
            jax.ShapeDtypeStruct((NP, Fn), jnp.float32),
        ],
    )(acc, hs, dinv, b, Wn)


def _tc_layer_last(acc, hs, dinv, b, Fin):
    def body(acc_ref, hs_ref, dinv_ref, b_ref, x_ref):
        a = acc_ref[0] + acc_ref[1] + hs_ref[...]
        dinv = dinv_ref[...]
        xv = jnp.tanh(dinv * a + b_ref[...])
        x_ref[...] = jnp.where(dinv > 0, xv, 0.0)

    return pl.pallas_call(
        body,
        grid=(NP // _BR,),
        in_specs=[
            pl.BlockSpec((NC, _BR, Fin), lambda i: (0, i, 0)),
            pl.BlockSpec((_BR, Fin), lambda i: (i, 0)),
            pl.BlockSpec((_BR, 1), lambda i: (i, 0)),
            pl.BlockSpec((1, Fin), lambda i: (0, 0)),
        ],
        out_specs=pl.BlockSpec((_BR, Fin), lambda i: (i, 0)),
        out_shape=jax.ShapeDtypeStruct((NP, Fin), jnp.float32),
    )(acc, hs, dinv, b)


def _tc_head(sel, W5p, b5, W6, b6, fc1p, fc1b, fc2, fc2b):
    def body(s_ref, w5_ref, b5_ref, w6_ref, b6_ref, f1_ref, f1b_ref,
             f2_ref, f2b_ref, o_ref):
        c5 = jnp.maximum(
            jnp.dot(s_ref[...], w5_ref[...],
                    preferred_element_type=jnp.float32) + b5_ref[...], 0.0)
        c4 = c5.reshape(NG, 16, 2, 16)
        p = jnp.max(c4, axis=2)
        w6 = w6_ref[...]
        b6 = b6_ref[...]
        outs = []
        for l in range(11):
            a = jnp.zeros((NG, 32), jnp.float32)
            for t in range(5):
                a = a + jnp.dot(p[:, l + t, :], w6[16 * t:16 * t + 16, :],
                                preferred_element_type=jnp.float32)
            outs.append(jnp.maximum(a + b6, 0.0))
        h6 = jnp.concatenate(outs, axis=1)
        h = jnp.maximum(
            jnp.dot(h6, f1_ref[...],
                    preferred_element_type=jnp.float32) + f1b_ref[...], 0.0)
        logits = jnp.dot(h, f2_ref[...],
                         preferred_element_type=jnp.float32) + f2b_ref[...]
        m = jnp.max(logits, axis=1, keepdims=True)
        lse = jnp.log(jnp.sum(jnp.exp(logits - m), axis=1, keepdims=True)) + m
        o_ref[...] = logits - lse

    return pl.pallas_call(
        body,
        out_shape=jax.ShapeDtypeStruct((NG, 10), jnp.float32),
    )(sel, W5p, b5, W6, b6, fc1p, fc1b, fc2, fc2b)


def kernel(x, edge_index, batch, W1, b1, W2, b2, W3, b3, W4, b4,
           conv5_w, conv5_b, conv6_w, conv6_b, fc1_w, fc1_b, fc2_w, fc2_b):
    src = jnp.pad(edge_index[0], (0, EPAD - E)).reshape(NW, CPW, 128)
    dst = jnp.pad(edge_index[1], (0, EPAD - E)).reshape(NW, CPW, 128)
    x_pad = jnp.pad(x, ((0, NP - N), (0, 0)))

    deg_part, srcp, bounds = _sc_prep_kernel()(src, dst, batch)
    dinv, h1s = _tc_prep(deg_part.reshape(NC, NP, 1), x_pad, W1)

    srcp_g = srcp.reshape(NW, CPW // GRP, GRP * 128)
    dst_g = dst.reshape(NW, CPW // GRP, GRP * 128)
    _scatter32 = _make_scatter(32)
    acc1 = _scatter32(h1s, srcp_g, dst_g)
    x1, h2s = _tc_layer(acc1, h1s, dinv, b1.reshape(1, 32), W2, 32, 32)
    acc2 = _scatter32(h2s, srcp_g, dst_g)
    x2, h3s = _tc_layer(acc2, h2s, dinv, b2.reshape(1, 32), W3, 32, 32)
    acc3 = _scatter32(h3s, srcp_g, dst_g)
    W4p = jnp.pad(W4, ((0, 0), (0, 7)))
    x3, h4s = _tc_layer(acc3, h3s, dinv, b3.reshape(1, 32), W4p, 32, 8)
    acc4 = _make_scatter(8)(h4s, srcp_g, dst_g)
    b4p = jnp.pad(b4, (0, 7)).reshape(1, 8)
    x4 = _tc_layer_last(acc4, h4s, dinv, b4p, 8)

    xc = jnp.concatenate(
        [x1, x2, x3, x4[:, :1], jnp.zeros((NP, 31), jnp.float32)], axis=1)
    keys = x4[:, 0]

    sel = _sc_sortpool_kernel()(keys, bounds, xc)

    W5p = jnp.pad(conv5_w[:, 0, :].T, ((0, 31), (0, 0)))
    W6 = conv6_w.transpose(2, 1, 0).reshape(80, 32)
    fc1p = fc1_w.reshape(32, 11, 128).transpose(1, 0, 2).reshape(352, 128)
    return _tc_head(sel, W5p, conv5_b.reshape(1, 16), W6,
                    conv6_b.reshape(1, 32), fc1p, fc1_b.reshape(1, 128),
                    fc2_w, fc2_b.reshape(1, 10))


# speedup vs baseline: 2.1744x; 1.0827x over previous
import functools

import jax
import jax.numpy as jnp
from jax import lax
from jax.experimental import pallas as pl
from jax.experimental.pallas import tpu as pltpu
from jax.experimental.pallas import tpu_sc as plsc

N = 10000
NP = 10240
E = 320000
F0 = 128
NG = 64
K = 30
NC = 2
NS = 16
NW = NC * NS
CPW = 80
EPT = CPW * 128
EPAD = NW * EPT
RPT = NP // NS
GRP = 4
DUMMY = N

def _mesh():
    return plsc.VectorSubcoreMesh(core_axis_name="c", subcore_axis_name="s",
                                  num_cores=NC, num_subcores=NS)


@functools.cache
def _sc_prep_kernel():
    return functools.partial(
        pl.kernel,
        mesh=_mesh(),
        compiler_params=pltpu.CompilerParams(needs_layout_passes=False),
        out_type=[
            jax.ShapeDtypeStruct((NC, NP), jnp.float32),
            jax.ShapeDtypeStruct((NW, CPW, 128), jnp.int32),
            jax.ShapeDtypeStruct((128,), jnp.int32),
        ],
        scratch_types=[
            pltpu.VMEM((CPW, 128), jnp.int32),
            pltpu.VMEM((CPW, 128), jnp.int32),
            pltpu.VMEM((CPW, 128), jnp.int32),
            pltpu.VMEM((NP,), jnp.float32),
            pltpu.VMEM_SHARED((NS, NP), jnp.float32),
            pltpu.VMEM((NS, RPT), jnp.float32),
            pltpu.VMEM((RPT,), jnp.float32),
            pltpu.VMEM((N,), jnp.int32),
            pltpu.VMEM((64,), jnp.int32),
            pltpu.VMEM((128,), jnp.int32),
        ],
    )(_sc_prep_body)


def _sc_prep_body(src_hbm, dst_hbm, batch_hbm, deg_out, srcp_out, bounds_out,
             src_v, dst_v, srcp_v, deg_v, shared_deg, part_v, red_v,
             batch_v, counts_v, bounds_v):
    cid = lax.axis_index("c")
    sid = lax.axis_index("s")
    wid = sid * NC + cid

    zf = jnp.zeros((16,), jnp.float32)

    def _zero(i, _):
        deg_v[pl.ds(i * 16, 16)] = zf
        return 0
    lax.fori_loop(0, NP // 16, _zero, 0)

    pltpu.sync_copy(src_hbm.at[wid], src_v)
    pltpu.sync_copy(dst_hbm.at[wid], dst_v)

    dummy16 = jnp.full((16,), DUMMY, jnp.int32)

    def _edges(j, _):
        for k in range(8):
            s = src_v[j, pl.ds(k * 16, 16)]
            d = dst_v[j, pl.ds(k * 16, 16)]
            m = s != d
            plsc.addupdate_scatter(
                deg_v, [d], jnp.where(m, 1.0, 0.0).astype(jnp.float32))
            srcp_v[j, pl.ds(k * 16, 16)] = jnp.where(m, s, dummy16)
        return 0
    lax.fori_loop(0, CPW, _edges, 0)
    pltpu.sync_copy(srcp_v, srcp_out.at[wid])

    pltpu.sync_copy(deg_v, shared_deg.at[sid])
    plsc.subcore_barrier()
    pltpu.sync_copy(shared_deg.at[:, pl.ds(sid * RPT, RPT)], part_v)

    def _red(i, _):
        acc = jnp.zeros((16,), jnp.float32)
        for k in range(NS):
            acc = acc + part_v[k, pl.ds(i * 16, 16)]
        red_v[pl.ds(i * 16, 16)] = acc
        return 0
    lax.fori_loop(0, RPT // 16, _red, 0)
    pltpu.sync_copy(red_v, deg_out.at[cid, pl.ds(sid * RPT, RPT)])

    @pl.when(jnp.logical_and(cid == 0, sid == 0))
    def _bounds():
        pltpu.sync_copy(batch_hbm, batch_v)
        zi = jnp.zeros((16,), jnp.int32)
        for i in range(4):
            counts_v[pl.ds(i * 16, 16)] = zi
        ones_i = jnp.ones((16,), jnp.int32)

        def _cnt(i, _):
            b = batch_v[pl.ds(i * 16, 16)]
            plsc.addupdate_scatter(counts_v, [b], ones_i)
            return 0
        lax.fori_loop(0, N // 16, _cnt, 0)

        carry = jnp.int32(0)
        for g in range(4):
            c = counts_v[pl.ds(g * 16, 16)]
            cs = plsc.cumsum(c)
            bounds_v[pl.ds(g * 16, 16)] = carry + cs - c
            bounds_v[pl.ds(64 + g * 16, 16)] = carry + cs
            carry = carry + jnp.sum(c)
        pltpu.sync_copy(bounds_v, bounds_out)


@functools.cache
def _make_scatter(F):
    @functools.partial(
        pl.kernel,
        mesh=_mesh(),
        compiler_params=pltpu.CompilerParams(needs_layout_passes=False,
                                             use_tc_tiling_on_sc=False),
        out_type=jax.ShapeDtypeStruct((NC, NP, F), jnp.float32),
        scratch_types=[
            pltpu.VMEM((CPW // GRP, GRP * 128), jnp.int32),
            pltpu.VMEM((CPW // GRP, GRP * 128), jnp.int32),
            pltpu.VMEM((GRP * 128, F), jnp.float32),
            pltpu.VMEM((GRP * 128, F), jnp.float32),
            pltpu.VMEM((RPT, F), jnp.float32),
            pltpu.VMEM_SHARED((NP, F), jnp.float32),
            pltpu.VMEM_SHARED((NP, F), jnp.float32),
            pltpu.SemaphoreType.DMA,
        ],
    )
    def _scatter(hs_hbm, srcp_hbm, dst_hbm, acc_out,
                 sidx, didx, rows0, rows1, zbuf, acc_sh, hs_sh, sem):
        cid = lax.axis_index("c")
        sid = lax.axis_index("s")
        wid = sid * NC + cid

        zf = jnp.zeros((16,), jnp.float32)

        def _zero(i, _):
            for k in range(F // 16):
                zbuf[i, pl.ds(k * 16, 16)] = zf
            return 0
        lax.fori_loop(0, RPT, _zero, 0)
        pltpu.sync_copy(zbuf, acc_sh.at[pl.ds(sid * RPT, RPT), :])
        pltpu.sync_copy(hs_hbm.at[pl.ds(sid * RPT, RPT), :],
                        hs_sh.at[pl.ds(sid * RPT, RPT), :])

        pltpu.sync_copy(srcp_hbm.at[wid], sidx)
        pltpu.sync_copy(dst_hbm.at[wid], didx)
        plsc.subcore_barrier()

        nt = CPW // GRP
        pltpu.async_copy(hs_sh.at[sidx.at[0]], rows0, sem)

        def _edge_pair(j2, _):
            j0 = 2 * j2
            pltpu.make_async_copy(hs_sh.at[sidx.at[j0]], rows0, sem).wait()
            pltpu.async_copy(hs_sh.at[sidx.at[j0 + 1]], rows1, sem)
            pltpu.sync_copy(rows0, acc_sh.at[didx.at[j0]], add=True)
            pltpu.make_async_copy(
                hs_sh.at[sidx.at[j0 + 1]], rows1, sem).wait()

            @pl.when(j2 + 1 < nt // 2)
            def _prefetch():
                pltpu.async_copy(hs_sh.at[sidx.at[j0 + 2]], rows0, sem)

            pltpu.sync_copy(rows1, acc_sh.at[didx.at[j0 + 1]], add=True)
            return 0
        lax.fori_loop(0, nt // 2, _edge_pair, 0)

        plsc.subcore_barrier()
        pltpu.sync_copy(acc_sh.at[pl.ds(sid * RPT, RPT), :], zbuf)
        pltpu.sync_copy(zbuf, acc_out.at[cid, pl.ds(sid * RPT, RPT), :])

    return _scatter


GPT = NG // NW
_NEG_INF = float("-inf")
_IMAX = 2147483647


@functools.cache
def _sc_sortpool_kernel():
    return functools.partial(
        pl.kernel,
        mesh=_mesh(),
        compiler_params=pltpu.CompilerParams(needs_layout_passes=False),
        out_type=jax.ShapeDtypeStruct((NG * 32, 128), jnp.float32),
        scratch_types=[
            pltpu.VMEM((NP,), jnp.float32),
            pltpu.VMEM((160,), jnp.int32),
            pltpu.VMEM((32,), jnp.int32),
            pltpu.VMEM((32, 128), jnp.float32),
            pltpu.SemaphoreType.DMA,
        ],
    )(_sc_sortpool_body)


def _sc_sortpool_body(keys_hbm, bounds_hbm, xc_hbm, sel_out,
                 keys_v, bounds_v, idx_buf, rows, sem):
    cid = lax.axis_index("c")
    sid = lax.axis_index("s")
    wid = sid * NC + cid
    pltpu.sync_copy(keys_hbm, keys_v)
    pltpu.sync_copy(bounds_hbm, bounds_v.at[pl.ds(0, 128)])

    lane = jnp.arange(16, dtype=jnp.int32)
    neg16 = jnp.full((16,), _NEG_INF, jnp.float32)
    dummy16 = jnp.full((16,), DUMMY, jnp.int32)

    for gi in range(GPT):
        g = wid * GPT + gi
        start = bounds_v[pl.ds(g, 16)][0]
        end = bounds_v[pl.ds(64 + g, 16)][0]
        c_lo = start // 16
        c_hi = (end + 15) // 16

        res = [dummy16, dummy16]
        for r in range(K):
            def _scan(c, carry):
                m_v, i_v = carry
                base = c * 16
                kv = keys_v[pl.ds(base, 16)]
                gidx = base + lane
                valid = jnp.logical_and(gidx >= start, gidx < end)
                kv = jnp.where(valid, kv, neg16)
                upd = kv > m_v
                return jnp.where(upd, kv, m_v), jnp.where(upd, gidx, i_v)

            m_v, i_v = lax.fori_loop(
                c_lo, c_hi, _scan,
                (neg16, jnp.zeros((16,), jnp.int32)))
            m = jnp.max(m_v)
            idx = jnp.min(jnp.where(m_v == m, i_v, _IMAX))
            is_valid = m > _NEG_INF
            idx_final = jnp.where(is_valid, idx, DUMMY)
            plsc.store_scatter(
                keys_v, [jnp.full((16,), idx, jnp.int32)], neg16,
                mask=jnp.logical_and(lane == 0, is_valid))
            q, sl = divmod(r, 16)
            res[q] = jnp.where(lane == sl, idx_final, res[q])

        idx_buf[pl.ds(0, 16)] = res[0]
        idx_buf[pl.ds(16, 16)] = res[1]
        pltpu.async_copy(xc_hbm.at[idx_buf], rows, sem).wait()
        pltpu.sync_copy(rows, sel_out.at[pl.ds(g * 32, 32), :])


_BR = 1024


def _tc_prep(deg3, x_pad, W1):
    def body(deg_ref, x_ref, w_ref, dinv_ref, hs_ref):
        deg = deg_ref[0] + deg_ref[1]
        dinv = jnp.where(deg > 0, lax.rsqrt(deg), 0.0)
        dinv_ref[...] = dinv
        h = jnp.dot(x_ref[...], w_ref[...],
                    preferred_element_type=jnp.float32)
        hs_ref[...] = dinv * h

    return pl.pallas_call(
        body,
        grid=(NP // _BR,),
        in_specs=[
            pl.BlockSpec((NC, _BR, 1), lambda i: (0, i, 0)),
            pl.BlockSpec((_BR, F0), lambda i: (i, 0)),
            pl.BlockSpec((F0, 32), lambda i: (0, 0)),
        ],
        out_specs=[
            pl.BlockSpec((_BR, 1), lambda i: (i, 0)),
            pl.BlockSpec((_BR, 32), lambda i: (i, 0)),
        ],
        out_shape=[
            jax.ShapeDtypeStruct((NP, 1), jnp.float32),
            jax.ShapeDtypeStruct((NP, 32), jnp.float32),
        ],
    )(deg3, x_pad, W1)


def _tc_layer(acc, hs, dinv, b, Wn, Fin, Fn):
    def body(acc_ref, hs_ref, dinv_ref, b_ref, wn_ref, x_ref, hn_ref):
        a = acc_ref[0] + acc_ref[1] + hs_ref[...]
        dinv = dinv_ref[...]
        xv = jnp.tanh(dinv * a + b_ref[...])
        xv = jnp.where(dinv > 0, xv, 0.0)
        x_ref[...] = xv
        hn_ref[...] = dinv * jnp.dot(xv, wn_ref[...],
                                     preferred_element_type=jnp.float32)

    return pl.pallas_call(
        body,
        grid=(NP // _BR,),
        in_specs=[
            pl.BlockSpec((NC, _BR, Fin), lambda i: (0, i, 0)),
            pl.BlockSpec((_BR, Fin), lambda i: (i, 0)),
            pl.BlockSpec((_BR, 1), lambda i: (i, 0)),
            pl.BlockSpec((1, Fin), lambda i: (0, 0)),
            pl.BlockSpec((Fin, Fn), lambda i: (0, 0)),
        ],
        out_specs=[
            pl.BlockSpec((_BR, Fin), lambda i: (i, 0)),
            pl.BlockSpec((_BR, Fn), lambda i: (i, 0)),
        ],
        out_shape=[
            jax.ShapeDtypeStruct((NP, Fin), jnp.float32),
            jax.ShapeDtypeStruct((NP, Fn), jnp.float32),
        ],
    )(acc, hs, dinv, b, Wn)


def _tc_layer_last(acc, hs, dinv, b, Fin):
    def body(acc_ref, hs_ref, dinv_ref, b_ref, x_ref):
        a = acc_ref[0] + acc_ref[1] + hs_ref[...]
        dinv = dinv_ref[...]
        xv = jnp.tanh(dinv * a + b_ref[...])
        x_ref[...] = jnp.where(dinv > 0, xv, 0.0)

    return pl.pallas_call(
        body,
        grid=(NP // _BR,),
        in_specs=[
            pl.BlockSpec((NC, _BR, Fin), lambda i: (0, i, 0)),
            pl.BlockSpec((_BR, Fin), lambda i: (i, 0)),
            pl.BlockSpec((_BR, 1), lambda i: (i, 0)),
            pl.BlockSpec((1, Fin), lambda i: (0, 0)),
        ],
        out_specs=pl.BlockSpec((_BR, Fin), lambda i: (i, 0)),
        out_shape=jax.ShapeDtypeStruct((NP, Fin), jnp.float32),
    )(acc, hs, dinv, b)


def _tc_head(sel, W5p, b5, W6, b6, fc1p, fc1b, fc2, fc2b):
    def body(s_ref, w5_ref, b5_ref, w6_ref, b6_ref, f1_ref, f1b_ref,
             f2_ref, f2b_ref, o_ref):
        c5 = jnp.maximum(
            jnp.dot(s_ref[...], w5_ref[...],
                    preferred_element_type=jnp.float32) + b5_ref[...], 0.0)
        c4 = c5.reshape(NG, 16, 2, 16)
        p = jnp.max(c4, axis=2)
        w6 = w6_ref[...]
        b6 = b6_ref[...]
        outs = []
        for l in range(11):
            a = jnp.zeros((NG, 32), jnp.float32)
            for t in range(5):
                a = a + jnp.dot(p[:, l + t, :], w6[16 * t:16 * t + 16, :],
                                preferred_element_type=jnp.float32)
            outs.append(jnp.maximum(a + b6, 0.0))
        h6 = jnp.concatenate(outs, axis=1)
        h = jnp.maximum(
            jnp.dot(h6, f1_ref[...],
                    preferred_element_type=jnp.float32) + f1b_ref[...], 0.0)
        logits = jnp.dot(h, f2_ref[...],
                         preferred_element_type=jnp.float32) + f2b_ref[...]
        m = jnp.max(logits, axis=1, keepdims=True)
        lse = jnp.log(jnp.sum(jnp.exp(logits - m), axis=1, keepdims=True)) + m
        o_ref[...] = logits - lse

    return pl.pallas_call(
        body,
        out_shape=jax.ShapeDtypeStruct((NG, 10), jnp.float32),
    )(sel, W5p, b5, W6, b6, fc1p, fc1b, fc2, fc2b)


def kernel(x, edge_index, batch, W1, b1, W2, b2, W3, b3, W4, b4,
           conv5_w, conv5_b, conv6_w, conv6_b, fc1_w, fc1_b, fc2_w, fc2_b):
    src = jnp.pad(edge_index[0], (0, EPAD - E)).reshape(NW, CPW, 128)
    dst = jnp.pad(edge_index[1], (0, EPAD - E)).reshape(NW, CPW, 128)
    x_pad = jnp.pad(x, ((0, NP - N), (0, 0)))

    deg_part, srcp, bounds = _sc_prep_kernel()(src, dst, batch)
    dinv, h1s = _tc_prep(deg_part.reshape(NC, NP, 1), x_pad, W1)

    srcp_g = srcp.reshape(NW, CPW // GRP, GRP * 128)
    dst_g = dst.reshape(NW, CPW // GRP, GRP * 128)
    _scatter32 = _make_scatter(32)
    acc1 = _scatter32(h1s, srcp_g, dst_g)
    x1, h2s = _tc_layer(acc1, h1s, dinv, b1.reshape(1, 32), W2, 32, 32)
    acc2 = _scatter32(h2s, srcp_g, dst_g)
    x2, h3s = _tc_layer(acc2, h2s, dinv, b2.reshape(1, 32), W3, 32, 32)
    acc3 = _scatter32(h3s, srcp_g, dst_g)
    W4p = jnp.pad(W4, ((0, 0), (0, 7)))
    x3, h4s = _tc_layer(acc3, h3s, dinv, b3.reshape(1, 32), W4p, 32, 8)
    acc4 = _make_scatter(8)(h4s, srcp_g, dst_g)
    b4p = jnp.pad(b4, (0, 7)).reshape(1, 8)
    x4 = _tc_layer_last(acc4, h4s, dinv, b4p, 8)

    xc = jnp.concatenate(
        [x1, x2, x3, x4[:, :1], jnp.zeros((NP, 31), jnp.float32)], axis=1)
    keys = x4[:, 0]

    sel = _sc_sortpool_kernel()(keys, bounds, xc)

    W5p = jnp.pad(conv5_w[:, 0, :].T, ((0, 31), (0, 0)))
    W6 = conv6_w.transpose(2, 1, 0).reshape(80, 32)
    fc1p = fc1_w.reshape(32, 11, 128).transpose(1, 0, 2).reshape(352, 128)
    return _tc_head(sel, W5p, conv5_b.reshape(1, 16), W6,
                    conv6_b.reshape(1, 32), fc1p, fc1_b.reshape(1, 128),
                    fc2_w, fc2_b.reshape(1, 10))


# speedup vs baseline: 2.3935x; 1.1008x over previous
import functools

import jax
import jax.numpy as jnp
from jax import lax
from jax.experimental import pallas as pl
from jax.experimental.pallas import tpu as pltpu
from jax.experimental.pallas import tpu_sc as plsc

N = 10000
NP = 10240
E = 320000
F0 = 128
NG = 64
K = 30
NC = 2
NS = 16
NW = NC * NS
CPW = 80
EPT = CPW * 128
EPAD = NW * EPT
RPT = NP // NS
GRP = 4
DUMMY = N

def _mesh():
    return plsc.VectorSubcoreMesh(core_axis_name="c", subcore_axis_name="s",
                                  num_cores=NC, num_subcores=NS)


@functools.cache
def _sc_prep_kernel():
    return functools.partial(
        pl.kernel,
        mesh=_mesh(),
        compiler_params=pltpu.CompilerParams(needs_layout_passes=False),
        out_type=[
            jax.ShapeDtypeStruct((NC, NP), jnp.float32),
            jax.ShapeDtypeStruct((NW, CPW, 128), jnp.int32),
            jax.ShapeDtypeStruct((128,), jnp.int32),
        ],
        scratch_types=[
            pltpu.VMEM((CPW, 128), jnp.int32),
            pltpu.VMEM((CPW, 128), jnp.int32),
            pltpu.VMEM((CPW, 128), jnp.int32),
            pltpu.VMEM((NP,), jnp.float32),
            pltpu.VMEM_SHARED((NS, NP), jnp.float32),
            pltpu.VMEM((NS, RPT), jnp.float32),
            pltpu.VMEM((RPT,), jnp.float32),
            pltpu.VMEM((N,), jnp.int32),
            pltpu.VMEM((64,), jnp.int32),
            pltpu.VMEM((128,), jnp.int32),
        ],
    )(_sc_prep_body)


def _sc_prep_body(src_hbm, dst_hbm, batch_hbm, deg_out, srcp_out, bounds_out,
             src_v, dst_v, srcp_v, deg_v, shared_deg, part_v, red_v,
             batch_v, counts_v, bounds_v):
    cid = lax.axis_index("c")
    sid = lax.axis_index("s")
    wid = sid * NC + cid

    zf = jnp.zeros((16,), jnp.float32)

    def _zero(i, _):
        deg_v[pl.ds(i * 16, 16)] = zf
        return 0
    lax.fori_loop(0, NP // 16, _zero, 0)

    pltpu.sync_copy(src_hbm.at[wid], src_v)
    pltpu.sync_copy(dst_hbm.at[wid], dst_v)

    dummy16 = jnp.full((16,), DUMMY, jnp.int32)

    def _edges(j, _):
        for k in range(8):
            s = src_v[j, pl.ds(k * 16, 16)]
            d = dst_v[j, pl.ds(k * 16, 16)]
            m = s != d
            plsc.addupdate_scatter(
                deg_v, [d], jnp.where(m, 1.0, 0.0).astype(jnp.float32))
            srcp_v[j, pl.ds(k * 16, 16)] = jnp.where(m, s, dummy16)
        return 0
    lax.fori_loop(0, CPW, _edges, 0)
    pltpu.sync_copy(srcp_v, srcp_out.at[wid])

    pltpu.sync_copy(deg_v, shared_deg.at[sid])
    plsc.subcore_barrier()
    pltpu.sync_copy(shared_deg.at[:, pl.ds(sid * RPT, RPT)], part_v)

    def _red(i, _):
        acc = jnp.zeros((16,), jnp.float32)
        for k in range(NS):
            acc = acc + part_v[k, pl.ds(i * 16, 16)]
        red_v[pl.ds(i * 16, 16)] = acc
        return 0
    lax.fori_loop(0, RPT // 16, _red, 0)
    pltpu.sync_copy(red_v, deg_out.at[cid, pl.ds(sid * RPT, RPT)])

    @pl.when(jnp.logical_and(cid == 0, sid == 0))
    def _bounds():
        pltpu.sync_copy(batch_hbm, batch_v)
        zi = jnp.zeros((16,), jnp.int32)
        for i in range(4):
            counts_v[pl.ds(i * 16, 16)] = zi
        ones_i = jnp.ones((16,), jnp.int32)

        def _cnt(i, _):
            b = batch_v[pl.ds(i * 16, 16)]
            plsc.addupdate_scatter(counts_v, [b], ones_i)
            return 0
        lax.fori_loop(0, N // 16, _cnt, 0)

        carry = jnp.int32(0)
        for g in range(4):
            c = counts_v[pl.ds(g * 16, 16)]
            cs = plsc.cumsum(c)
            bounds_v[pl.ds(g * 16, 16)] = carry + cs - c
            bounds_v[pl.ds(64 + g * 16, 16)] = carry + cs
            carry = carry + jnp.sum(c)
        pltpu.sync_copy(bounds_v, bounds_out)


@functools.cache
def _make_scatter(F):
    @functools.partial(
        pl.kernel,
        mesh=_mesh(),
        compiler_params=pltpu.CompilerParams(needs_layout_passes=False,
                                             use_tc_tiling_on_sc=False),
        out_type=jax.ShapeDtypeStruct((NC, NP, F), jnp.float32),
        scratch_types=[
            pltpu.VMEM((CPW // GRP, GRP * 128), jnp.int32),
            pltpu.VMEM((CPW // GRP, GRP * 128), jnp.int32),
            pltpu.VMEM((GRP * 128, F), jnp.float32),
            pltpu.VMEM((GRP * 128, F), jnp.float32),
            pltpu.VMEM((RPT, F), jnp.float32),
            pltpu.VMEM_SHARED((NP, F), jnp.float32),
            pltpu.VMEM_SHARED((NP, F), jnp.float32),
            pltpu.SemaphoreType.DMA,
        ],
    )
    def _scatter(hs_hbm, srcp_hbm, dst_hbm, acc_out,
                 sidx, didx, rows0, rows1, zbuf, acc_sh, hs_sh, sem):
        cid = lax.axis_index("c")
        sid = lax.axis_index("s")
        wid = sid * NC + cid

        zf = jnp.zeros((16,), jnp.float32)

        def _zero(i, _):
            for k in range(F // 16):
                zbuf[i, pl.ds(k * 16, 16)] = zf
            return 0
        lax.fori_loop(0, RPT, _zero, 0)
        pltpu.sync_copy(zbuf, acc_sh.at[pl.ds(sid * RPT, RPT), :])
        pltpu.sync_copy(hs_hbm.at[pl.ds(sid * RPT, RPT), :],
                        hs_sh.at[pl.ds(sid * RPT, RPT), :])

        pltpu.sync_copy(srcp_hbm.at[wid], sidx)
        pltpu.sync_copy(dst_hbm.at[wid], didx)
        plsc.subcore_barrier()

        nt = CPW // GRP
        pltpu.async_copy(hs_sh.at[sidx.at[0]], rows0, sem)

        def _edge_pair(j2, _):
            j0 = 2 * j2
            pltpu.make_async_copy(hs_sh.at[sidx.at[j0]], rows0, sem).wait()
            pltpu.async_copy(hs_sh.at[sidx.at[j0 + 1]], rows1, sem)
            pltpu.sync_copy(rows0, acc_sh.at[didx.at[j0]], add=True)
            pltpu.make_async_copy(
                hs_sh.at[sidx.at[j0 + 1]], rows1, sem).wait()

            @pl.when(j2 + 1 < nt // 2)
            def _prefetch():
                pltpu.async_copy(hs_sh.at[sidx.at[j0 + 2]], rows0, sem)

            pltpu.sync_copy(rows1, acc_sh.at[didx.at[j0 + 1]], add=True)
            return 0
        lax.fori_loop(0, nt // 2, _edge_pair, 0)

        plsc.subcore_barrier()
        pltpu.sync_copy(acc_sh.at[pl.ds(sid * RPT, RPT), :], zbuf)
        pltpu.sync_copy(zbuf, acc_out.at[cid, pl.ds(sid * RPT, RPT), :])

    return _scatter


GPT = NG // NW
_NEG_INF = float("-inf")
_IMAX = 2147483647


@functools.cache
def _sc_sortpool_kernel():
    return functools.partial(
        pl.kernel,
        mesh=_mesh(),
        compiler_params=pltpu.CompilerParams(needs_layout_passes=False,
                                             use_tc_tiling_on_sc=False),
        out_type=jax.ShapeDtypeStruct((NG * 32, 128), jnp.float32),
        scratch_types=[
            pltpu.VMEM((RPT, 8), jnp.float32),
            pltpu.VMEM((RPT, 8), jnp.float32),
            pltpu.VMEM((RPT, 8), jnp.float32),
            pltpu.VMEM((RPT,), jnp.float32),
            pltpu.VMEM((RPT,), jnp.float32),
            pltpu.VMEM((16,), jnp.float32),
            pltpu.VMEM_SHARED((NP,), jnp.float32),
            pltpu.VMEM((NP,), jnp.float32),
            pltpu.VMEM((160,), jnp.int32),
            pltpu.VMEM((32,), jnp.int32),
            pltpu.VMEM((32, 128), jnp.float32),
            pltpu.SemaphoreType.DMA,
        ],
    )(_sc_sortpool_body)


def _sc_sortpool_body(acc4_hbm, h4s_hbm, dinv_hbm, b4_hbm, bounds_hbm, xc_hbm,
                      sel_out, a0_v, a1_v, hs_v, dv, kslice_v, b4_v,
                      keys_sh, keys_v, bounds_v, idx_buf, rows, sem):
    cid = lax.axis_index("c")
    sid = lax.axis_index("s")
    wid = sid * NC + cid
    r0 = sid * RPT

    pltpu.sync_copy(acc4_hbm.at[0, pl.ds(r0, RPT), :], a0_v)
    pltpu.sync_copy(acc4_hbm.at[1, pl.ds(r0, RPT), :], a1_v)
    pltpu.sync_copy(h4s_hbm.at[pl.ds(r0, RPT), :], hs_v)
    pltpu.sync_copy(dinv_hbm.at[pl.ds(r0, RPT)], dv)
    pltpu.sync_copy(b4_hbm, b4_v)
    pltpu.sync_copy(bounds_hbm, bounds_v.at[pl.ds(0, 128)])

    lane = jnp.arange(16, dtype=jnp.int32)
    zero16 = jnp.zeros((16,), jnp.int32)
    b4 = b4_v[...][0]

    def _keys(i, _):
        ridx = i * 16 + lane
        k0 = plsc.load_gather(a0_v, [ridx, zero16])
        k1 = plsc.load_gather(a1_v, [ridx, zero16])
        kh = plsc.load_gather(hs_v, [ridx, zero16])
        d = dv[pl.ds(i * 16, 16)]
        z = d * (k0 + k1 + kh) + b4
        kslice_v[pl.ds(i * 16, 16)] = jnp.where(d > 0, z, 0.0)
        return 0
    lax.fori_loop(0, RPT // 16, _keys, 0)

    pltpu.sync_copy(kslice_v, keys_sh.at[pl.ds(r0, RPT)])
    plsc.subcore_barrier()
    pltpu.sync_copy(keys_sh, keys_v)

    neg16 = jnp.full((16,), _NEG_INF, jnp.float32)
    dummy16 = jnp.full((16,), DUMMY, jnp.int32)

    for gi in range(GPT):
        g = wid * GPT + gi
        start = bounds_v[pl.ds(g, 16)][0]
        end = bounds_v[pl.ds(64 + g, 16)][0]
        c_lo = start // 16
        c_hi = (end + 15) // 16

        res = [dummy16, dummy16]
        resk = [jnp.zeros((16,), jnp.float32), jnp.zeros((16,), jnp.float32)]
        for r in range(K):
            def _scan(c, carry):
                m_v, i_v = carry
                base = c * 16
                kv = keys_v[pl.ds(base, 16)]
                gidx = base + lane
                valid = jnp.logical_and(gidx >= start, gidx < end)
                kv = jnp.where(valid, kv, neg16)
                upd = kv > m_v
                return jnp.where(upd, kv, m_v), jnp.where(upd, gidx, i_v)

            m_v, i_v = lax.fori_loop(
                c_lo, c_hi, _scan,
                (neg16, jnp.zeros((16,), jnp.int32)))
            m = jnp.max(m_v)
            idx = jnp.min(jnp.where(m_v == m, i_v, _IMAX))
            is_valid = m > _NEG_INF
            idx_final = jnp.where(is_valid, idx, DUMMY)
            plsc.store_scatter(
                keys_v, [jnp.full((16,), idx, jnp.int32)], neg16,
                mask=jnp.logical_and(lane == 0, is_valid))
            q, sl = divmod(r, 16)
            res[q] = jnp.where(lane == sl, idx_final, res[q])
            resk[q] = jnp.where(
                jnp.logical_and(lane == sl, is_valid), m, resk[q])

        idx_buf[pl.ds(0, 16)] = res[0]
        idx_buf[pl.ds(16, 16)] = res[1]
        pltpu.async_copy(xc_hbm.at[idx_buf], rows, sem).wait()
        for half in range(2):
            plsc.store_scatter(
                rows, [lane + 16 * half, jnp.full((16,), 96, jnp.int32)],
                resk[half])
        pltpu.sync_copy(rows, sel_out.at[pl.ds(g * 32, 32), :])


_BR = 1024


def _tc_prep(deg3, x_pad, W1):
    def body(deg_ref, x_ref, w_ref, dinv_ref, hs_ref):
        deg = deg_ref[0] + deg_ref[1]
        dinv = jnp.where(deg > 0, lax.rsqrt(deg), 0.0)
        dinv_ref[...] = dinv
        h = jnp.dot(x_ref[...], w_ref[...],
                    preferred_element_type=jnp.float32)
        hs_ref[...] = dinv * h

    return pl.pallas_call(
        body,
        grid=(NP // _BR,),
        in_specs=[
            pl.BlockSpec((NC, _BR, 1), lambda i: (0, i, 0)),
            pl.BlockSpec((_BR, F0), lambda i: (i, 0)),
            pl.BlockSpec((F0, 32), lambda i: (0, 0)),
        ],
        out_specs=[
            pl.BlockSpec((_BR, 1), lambda i: (i, 0)),
            pl.BlockSpec((_BR, 32), lambda i: (i, 0)),
        ],
        out_shape=[
            jax.ShapeDtypeStruct((NP, 1), jnp.float32),
            jax.ShapeDtypeStruct((NP, 32), jnp.float32),
        ],
    )(deg3, x_pad, W1)


def _tc_layer(acc, hs, dinv, b, Wn, Fin, Fn):
    def body(acc_ref, hs_ref, dinv_ref, b_ref, wn_ref, x_ref, hn_ref):
        a = acc_ref[0] + acc_ref[1] + hs_ref[...]
        dinv = dinv_ref[...]
        xv = jnp.tanh(dinv * a + b_ref[...])
        xv = jnp.where(dinv > 0, xv, 0.0)
        x_ref[...] = xv
        hn_ref[...] = dinv * jnp.dot(xv, wn_ref[...],
                                     preferred_element_type=jnp.float32)

    return pl.pallas_call(
        body,
        grid=(NP // _BR,),
        in_specs=[
            pl.BlockSpec((NC, _BR, Fin), lambda i: (0, i, 0)),
            pl.BlockSpec((_BR, Fin), lambda i: (i, 0)),
            pl.BlockSpec((_BR, 1), lambda i: (i, 0)),
            pl.BlockSpec((1, Fin), lambda i: (0, 0)),
            pl.BlockSpec((Fin, Fn), lambda i: (0, 0)),
        ],
        out_specs=[
            pl.BlockSpec((_BR, Fin), lambda i: (i, 0)),
            pl.BlockSpec((_BR, Fn), lambda i: (i, 0)),
        ],
        out_shape=[
            jax.ShapeDtypeStruct((NP, Fin), jnp.float32),
            jax.ShapeDtypeStruct((NP, Fn), jnp.float32),
        ],
    )(acc, hs, dinv, b, Wn)


def _tc_layer_last(acc, hs, dinv, b, Fin):
    def body(acc_ref, hs_ref, dinv_ref, b_ref, x_ref):
        a = acc_ref[0] + acc_ref[1] + hs_ref[...]
        dinv = dinv_ref[...]
        xv = jnp.tanh(dinv * a + b_ref[...])
        x_ref[...] = jnp.where(dinv > 0, xv, 0.0)

    return pl.pallas_call(
        body,
        grid=(NP // _BR,),
        in_specs=[
            pl.BlockSpec((NC, _BR, Fin), lambda i: (0, i, 0)),
            pl.BlockSpec((_BR, Fin), lambda i: (i, 0)),
            pl.BlockSpec((_BR, 1), lambda i: (i, 0)),
            pl.BlockSpec((1, Fin), lambda i: (0, 0)),
        ],
        out_specs=pl.BlockSpec((_BR, Fin), lambda i: (i, 0)),
        out_shape=jax.ShapeDtypeStruct((NP, Fin), jnp.float32),
    )(acc, hs, dinv, b)


def _tc_head(sel, W5p, b5, W6, b6, fc1p, fc1b, fc2, fc2b):
    def body(s_ref, w5_ref, b5_ref, w6_ref, b6_ref, f1_ref, f1b_ref,
             f2_ref, f2b_ref, o_ref):
        s = s_ref[...]
        s = jnp.concatenate(
            [s[:, :96], jnp.tanh(s[:, 96:97]), s[:, 97:]], axis=1)
        c5 = jnp.maximum(
            jnp.dot(s, w5_ref[...],
                    preferred_element_type=jnp.float32) + b5_ref[...], 0.0)
        c4 = c5.reshape(NG, 16, 2, 16)
        p = jnp.max(c4, axis=2)
        w6 = w6_ref[...]
        b6 = b6_ref[...]
        outs = []
        for l in range(11):
            a = jnp.zeros((NG, 32), jnp.float32)
            for t in range(5):
                a = a + jnp.dot(p[:, l + t, :], w6[16 * t:16 * t + 16, :],
                                preferred_element_type=jnp.float32)
            outs.append(jnp.maximum(a + b6, 0.0))
        h6 = jnp.concatenate(outs, axis=1)
        h = jnp.maximum(
            jnp.dot(h6, f1_ref[...],
                    preferred_element_type=jnp.float32) + f1b_ref[...], 0.0)
        logits = jnp.dot(h, f2_ref[...],
                         preferred_element_type=jnp.float32) + f2b_ref[...]
        m = jnp.max(logits, axis=1, keepdims=True)
        lse = jnp.log(jnp.sum(jnp.exp(logits - m), axis=1, keepdims=True)) + m
        o_ref[...] = logits - lse

    return pl.pallas_call(
        body,
        out_shape=jax.ShapeDtypeStruct((NG, 10), jnp.float32),
    )(sel, W5p, b5, W6, b6, fc1p, fc1b, fc2, fc2b)


def kernel(x, edge_index, batch, W1, b1, W2, b2, W3, b3, W4, b4,
           conv5_w, conv5_b, conv6_w, conv6_b, fc1_w, fc1_b, fc2_w, fc2_b):
    src = jnp.pad(edge_index[0], (0, EPAD - E)).reshape(NW, CPW, 128)
    dst = jnp.pad(edge_index[1], (0, EPAD - E)).reshape(NW, CPW, 128)
    x_pad = jnp.pad(x, ((0, NP - N), (0, 0)))

    deg_part, srcp, bounds = _sc_prep_kernel()(src, dst, batch)
    dinv, h1s = _tc_prep(deg_part.reshape(NC, NP, 1), x_pad, W1)

    srcp_g = srcp.reshape(NW, CPW // GRP, GRP * 128)
    dst_g = dst.reshape(NW, CPW // GRP, GRP * 128)
    _scatter32 = _make_scatter(32)
    acc1 = _scatter32(h1s, srcp_g, dst_g)
    x1, h2s = _tc_layer(acc1, h1s, dinv, b1.reshape(1, 32), W2, 32, 32)
    acc2 = _scatter32(h2s, srcp_g, dst_g)
    x2, h3s = _tc_layer(acc2, h2s, dinv, b2.reshape(1, 32), W3, 32, 32)
    acc3 = _scatter32(h3s, srcp_g, dst_g)
    W4p = jnp.pad(W4, ((0, 0), (0, 7)))
    x3, h4s = _tc_layer(acc3, h3s, dinv, b3.reshape(1, 32), W4p, 32, 8)
    acc4 = _make_scatter(8)(h4s, srcp_g, dst_g)

    xc = jnp.concatenate(
        [x1, x2, x3, jnp.zeros((NP, 32), jnp.float32)], axis=1)
    b4p16 = jnp.pad(b4, (0, 15))
    sel = _sc_sortpool_kernel()(acc4, h4s, dinv.reshape(NP), b4p16,
                                bounds, xc)

    W5p = jnp.pad(conv5_w[:, 0, :].T, ((0, 31), (0, 0)))
    W6 = conv6_w.transpose(2, 1, 0).reshape(80, 32)
    fc1p = fc1_w.reshape(32, 11, 128).transpose(1, 0, 2).reshape(352, 128)
    return _tc_head(sel, W5p, conv5_b.reshape(1, 16), W6,
                    conv6_b.reshape(1, 32), fc1p, fc1_b.reshape(1, 128),
                    fc2_w, fc2_b.reshape(1, 10))


# speedup vs baseline: 2.4005x; 1.0029x over previous
import functools

import jax
import jax.numpy as jnp
from jax import lax
from jax.experimental import pallas as pl
from jax.experimental.pallas import tpu as pltpu
from jax.experimental.pallas import tpu_sc as plsc

N = 10000
NP = 10240
E = 320000
F0 = 128
NG = 64
K = 30
NC = 2
NS = 16
NW = NC * NS
CPW = 80
EPT = CPW * 128
EPAD = NW * EPT
RPT = NP // NS
GRP = 5
DUMMY = N

def _mesh():
    return plsc.VectorSubcoreMesh(core_axis_name="c", subcore_axis_name="s",
                                  num_cores=NC, num_subcores=NS)


@functools.cache
def _sc_prep_kernel():
    return functools.partial(
        pl.kernel,
        mesh=_mesh(),
        compiler_params=pltpu.CompilerParams(needs_layout_passes=False),
        out_type=[
            jax.ShapeDtypeStruct((NC, NP), jnp.float32),
            jax.ShapeDtypeStruct((NW, CPW, 128), jnp.int32),
            jax.ShapeDtypeStruct((128,), jnp.int32),
        ],
        scratch_types=[
            pltpu.VMEM((CPW, 128), jnp.int32),
            pltpu.VMEM((CPW, 128), jnp.int32),
            pltpu.VMEM((CPW, 128), jnp.int32),
            pltpu.VMEM((NP,), jnp.float32),
            pltpu.VMEM_SHARED((NS, NP), jnp.float32),
            pltpu.VMEM((NS, RPT), jnp.float32),
            pltpu.VMEM((RPT,), jnp.float32),
            pltpu.VMEM((N,), jnp.int32),
            pltpu.VMEM((64,), jnp.int32),
            pltpu.VMEM((128,), jnp.int32),
        ],
    )(_sc_prep_body)


def _sc_prep_body(src_hbm, dst_hbm, batch_hbm, deg_out, srcp_out, bounds_out,
             src_v, dst_v, srcp_v, deg_v, shared_deg, part_v, red_v,
             batch_v, counts_v, bounds_v):
    cid = lax.axis_index("c")
    sid = lax.axis_index("s")
    wid = sid * NC + cid

    zf = jnp.zeros((16,), jnp.float32)

    def _zero(i, _):
        deg_v[pl.ds(i * 16, 16)] = zf
        return 0
    lax.fori_loop(0, NP // 16, _zero, 0)

    pltpu.sync_copy(src_hbm.at[wid], src_v)
    pltpu.sync_copy(dst_hbm.at[wid], dst_v)

    dummy16 = jnp.full((16,), DUMMY, jnp.int32)

    def _edges(j, _):
        for k in range(8):
            s = src_v[j, pl.ds(k * 16, 16)]
            d = dst_v[j, pl.ds(k * 16, 16)]
            m = s != d
            plsc.addupdate_scatter(
                deg_v, [d], jnp.where(m, 1.0, 0.0).astype(jnp.float32))
            srcp_v[j, pl.ds(k * 16, 16)] = jnp.where(m, s, dummy16)
        return 0
    lax.fori_loop(0, CPW, _edges, 0)
    pltpu.sync_copy(srcp_v, srcp_out.at[wid])

    pltpu.sync_copy(deg_v, shared_deg.at[sid])
    plsc.subcore_barrier()
    pltpu.sync_copy(shared_deg.at[:, pl.ds(sid * RPT, RPT)], part_v)

    def _red(i, _):
        acc = jnp.zeros((16,), jnp.float32)
        for k in range(NS):
            acc = acc + part_v[k, pl.ds(i * 16, 16)]
        red_v[pl.ds(i * 16, 16)] = acc
        return 0
    lax.fori_loop(0, RPT // 16, _red, 0)
    pltpu.sync_copy(red_v, deg_out.at[cid, pl.ds(sid * RPT, RPT)])

    @pl.when(jnp.logical_and(cid == 0, sid == 0))
    def _bounds():
        pltpu.sync_copy(batch_hbm, batch_v)
        zi = jnp.zeros((16,), jnp.int32)
        for i in range(4):
            counts_v[pl.ds(i * 16, 16)] = zi
        ones_i = jnp.ones((16,), jnp.int32)

        def _cnt(i, _):
            b = batch_v[pl.ds(i * 16, 16)]
            plsc.addupdate_scatter(counts_v, [b], ones_i)
            return 0
        lax.fori_loop(0, N // 16, _cnt, 0)

        carry = jnp.int32(0)
        for g in range(4):
            c = counts_v[pl.ds(g * 16, 16)]
            cs = plsc.cumsum(c)
            bounds_v[pl.ds(g * 16, 16)] = carry + cs - c
            bounds_v[pl.ds(64 + g * 16, 16)] = carry + cs
            carry = carry + jnp.sum(c)
        pltpu.sync_copy(bounds_v, bounds_out)


@functools.cache
def _make_scatter(F):
    @functools.partial(
        pl.kernel,
        mesh=_mesh(),
        compiler_params=pltpu.CompilerParams(needs_layout_passes=False,
                                             use_tc_tiling_on_sc=False),
        out_type=jax.ShapeDtypeStruct((NC, NP, F), jnp.float32),
        scratch_types=[
            pltpu.VMEM((CPW // GRP, GRP * 128), jnp.int32),
            pltpu.VMEM((CPW // GRP, GRP * 128), jnp.int32),
            pltpu.VMEM((GRP * 128, F), jnp.float32),
            pltpu.VMEM((GRP * 128, F), jnp.float32),
            pltpu.VMEM((RPT, F), jnp.float32),
            pltpu.VMEM_SHARED((NP, F), jnp.float32),
            pltpu.VMEM_SHARED((NP, F), jnp.float32),
            pltpu.SemaphoreType.DMA,
        ],
    )
    def _scatter(hs_hbm, srcp_hbm, dst_hbm, acc_out,
                 sidx, didx, rows0, rows1, zbuf, acc_sh, hs_sh, sem):
        cid = lax.axis_index("c")
        sid = lax.axis_index("s")
        wid = sid * NC + cid

        zf = jnp.zeros((16,), jnp.float32)

        def _zero(i, _):
            for k in range(F // 16):
                zbuf[i, pl.ds(k * 16, 16)] = zf
            return 0
        lax.fori_loop(0, RPT, _zero, 0)
        pltpu.sync_copy(zbuf, acc_sh.at[pl.ds(sid * RPT, RPT), :])
        pltpu.sync_copy(hs_hbm.at[pl.ds(sid * RPT, RPT), :],
                        hs_sh.at[pl.ds(sid * RPT, RPT), :])

        pltpu.sync_copy(srcp_hbm.at[wid], sidx)
        pltpu.sync_copy(dst_hbm.at[wid], didx)
        plsc.subcore_barrier()

        nt = CPW // GRP
        pltpu.async_copy(hs_sh.at[sidx.at[0]], rows0, sem)

        def _edge_pair(j2, _):
            j0 = 2 * j2
            pltpu.make_async_copy(hs_sh.at[sidx.at[j0]], rows0, sem).wait()
            pltpu.async_copy(hs_sh.at[sidx.at[j0 + 1]], rows1, sem)
            pltpu.sync_copy(rows0, acc_sh.at[didx.at[j0]], add=True)
            pltpu.make_async_copy(
                hs_sh.at[sidx.at[j0 + 1]], rows1, sem).wait()

            @pl.when(j2 + 1 < nt // 2)
            def _prefetch():
                pltpu.async_copy(hs_sh.at[sidx.at[j0 + 2]], rows0, sem)

            pltpu.sync_copy(rows1, acc_sh.at[didx.at[j0 + 1]], add=True)
            return 0
        lax.fori_loop(0, nt // 2, _edge_pair, 0)

        plsc.subcore_barrier()
        pltpu.sync_copy(acc_sh.at[pl.ds(sid * RPT, RPT), :], zbuf)
        pltpu.sync_copy(zbuf, acc_out.at[cid, pl.ds(sid * RPT, RPT), :])

    return _scatter


GPT = NG // NW
_NEG_INF = float("-inf")
_IMAX = 2147483647


@functools.cache
def _sc_sortpool_kernel():
    return functools.partial(
        pl.kernel,
        mesh=_mesh(),
        compiler_params=pltpu.CompilerParams(needs_layout_passes=False,
                                             use_tc_tiling_on_sc=False),
        out_type=jax.ShapeDtypeStruct((NG * 32, 128), jnp.float32),
        scratch_types=[
            pltpu.VMEM((RPT, 8), jnp.float32),
            pltpu.VMEM((RPT, 8), jnp.float32),
            pltpu.VMEM((RPT, 8), jnp.float32),
            pltpu.VMEM((RPT,), jnp.float32),
            pltpu.VMEM((RPT,), jnp.float32),
            pltpu.VMEM((16,), jnp.float32),
            pltpu.VMEM_SHARED((NP,), jnp.float32),
            pltpu.VMEM((NP,), jnp.float32),
            pltpu.VMEM((160,), jnp.int32),
            pltpu.VMEM((32,), jnp.int32),
            pltpu.VMEM((32, 128), jnp.float32),
            pltpu.SemaphoreType.DMA,
        ],
    )(_sc_sortpool_body)


def _sc_sortpool_body(acc4_hbm, h4s_hbm, dinv_hbm, b4_hbm, bounds_hbm, xc_hbm,
                      sel_out, a0_v, a1_v, hs_v, dv, kslice_v, b4_v,
                      keys_sh, keys_v, bounds_v, idx_buf, rows, sem):
    cid = lax.axis_index("c")
    sid = lax.axis_index("s")
    wid = sid * NC + cid
    r0 = sid * RPT

    pltpu.sync_copy(acc4_hbm.at[0, pl.ds(r0, RPT), :], a0_v)
    pltpu.sync_copy(acc4_hbm.at[1, pl.ds(r0, RPT), :], a1_v)
    pltpu.sync_copy(h4s_hbm.at[pl.ds(r0, RPT), :], hs_v)
    pltpu.sync_copy(dinv_hbm.at[pl.ds(r0, RPT)], dv)
    pltpu.sync_copy(b4_hbm, b4_v)
    pltpu.sync_copy(bounds_hbm, bounds_v.at[pl.ds(0, 128)])

    lane = jnp.arange(16, dtype=jnp.int32)
    zero16 = jnp.zeros((16,), jnp.int32)
    b4 = b4_v[...][0]

    def _keys(i, _):
        ridx = i * 16 + lane
        k0 = plsc.load_gather(a0_v, [ridx, zero16])
        k1 = plsc.load_gather(a1_v, [ridx, zero16])
        kh = plsc.load_gather(hs_v, [ridx, zero16])
        d = dv[pl.ds(i * 16, 16)]
        z = d * (k0 + k1 + kh) + b4
        kslice_v[pl.ds(i * 16, 16)] = jnp.where(d > 0, z, 0.0)
        return 0
    lax.fori_loop(0, RPT // 16, _keys, 0)

    pltpu.sync_copy(kslice_v, keys_sh.at[pl.ds(r0, RPT)])
    plsc.subcore_barrier()
    pltpu.sync_copy(keys_sh, keys_v)

    neg16 = jnp.full((16,), _NEG_INF, jnp.float32)
    dummy16 = jnp.full((16,), DUMMY, jnp.int32)

    for gi in range(GPT):
        g = wid * GPT + gi
        start = bounds_v[pl.ds(g, 16)][0]
        end = bounds_v[pl.ds(64 + g, 16)][0]
        c_lo = start // 16
        c_hi = (end + 15) // 16

        res = [dummy16, dummy16]
        resk = [jnp.zeros((16,), jnp.float32), jnp.zeros((16,), jnp.float32)]
        for r in range(K):
            def _scan(c, carry):
                m_v, i_v = carry
                base = c * 16
                kv = keys_v[pl.ds(base, 16)]
                gidx = base + lane
                valid = jnp.logical_and(gidx >= start, gidx < end)
                kv = jnp.where(valid, kv, neg16)
                upd = kv > m_v
                return jnp.where(upd, kv, m_v), jnp.where(upd, gidx, i_v)

            m_v, i_v = lax.fori_loop(
                c_lo, c_hi, _scan,
                (neg16, jnp.zeros((16,), jnp.int32)))
            m = jnp.max(m_v)
            idx = jnp.min(jnp.where(m_v == m, i_v, _IMAX))
            is_valid = m > _NEG_INF
            idx_final = jnp.where(is_valid, idx, DUMMY)
            plsc.store_scatter(
                keys_v, [jnp.full((16,), idx, jnp.int32)], neg16,
                mask=jnp.logical_and(lane == 0, is_valid))
            q, sl = divmod(r, 16)
            res[q] = jnp.where(lane == sl, idx_final, res[q])
            resk[q] = jnp.where(
                jnp.logical_and(lane == sl, is_valid), m, resk[q])

        idx_buf[pl.ds(0, 16)] = res[0]
        idx_buf[pl.ds(16, 16)] = res[1]
        pltpu.async_copy(xc_hbm.at[idx_buf], rows, sem).wait()
        for half in range(2):
            plsc.store_scatter(
                rows, [lane + 16 * half, jnp.full((16,), 96, jnp.int32)],
                resk[half])
        pltpu.sync_copy(rows, sel_out.at[pl.ds(g * 32, 32), :])


_BR = 1024


def _tc_prep(deg3, x_pad, W1):
    def body(deg_ref, x_ref, w_ref, dinv_ref, hs_ref):
        deg = deg_ref[0] + deg_ref[1]
        dinv = jnp.where(deg > 0, lax.rsqrt(deg), 0.0)
        dinv_ref[...] = dinv
        h = jnp.dot(x_ref[...], w_ref[...],
                    preferred_element_type=jnp.float32)
        hs_ref[...] = dinv * h

    return pl.pallas_call(
        body,
        grid=(NP // _BR,),
        in_specs=[
            pl.BlockSpec((NC, _BR, 1), lambda i: (0, i, 0)),
            pl.BlockSpec((_BR, F0), lambda i: (i, 0)),
            pl.BlockSpec((F0, 32), lambda i: (0, 0)),
        ],
        out_specs=[
            pl.BlockSpec((_BR, 1), lambda i: (i, 0)),
            pl.BlockSpec((_BR, 32), lambda i: (i, 0)),
        ],
        out_shape=[
            jax.ShapeDtypeStruct((NP, 1), jnp.float32),
            jax.ShapeDtypeStruct((NP, 32), jnp.float32),
        ],
    )(deg3, x_pad, W1)


def _tc_layer(acc, hs, dinv, b, Wn, Fin, Fn):
    def body(acc_ref, hs_ref, dinv_ref, b_ref, wn_ref, x_ref, hn_ref):
        a = acc_ref[0] + acc_ref[1] + hs_ref[...]
        dinv = dinv_ref[...]
        xv = jnp.tanh(dinv * a + b_ref[...])
        xv = jnp.where(dinv > 0, xv, 0.0)
        x_ref[...] = xv
        hn_ref[...] = dinv * jnp.dot(xv, wn_ref[...],
                                     preferred_element_type=jnp.float32)

    return pl.pallas_call(
        body,
        grid=(NP // _BR,),
        in_specs=[
            pl.BlockSpec((NC, _BR, Fin), lambda i: (0, i, 0)),
            pl.BlockSpec((_BR, Fin), lambda i: (i, 0)),
            pl.BlockSpec((_BR, 1), lambda i: (i, 0)),
            pl.BlockSpec((1, Fin), lambda i: (0, 0)),
            pl.BlockSpec((Fin, Fn), lambda i: (0, 0)),
        ],
        out_specs=[
            pl.BlockSpec((_BR, Fin), lambda i: (i, 0)),
            pl.BlockSpec((_BR, Fn), lambda i: (i, 0)),
        ],
        out_shape=[
            jax.ShapeDtypeStruct((NP, Fin), jnp.float32),
            jax.ShapeDtypeStruct((NP, Fn), jnp.float32),
        ],
    )(acc, hs, dinv, b, Wn)


def _tc_layer_last(acc, hs, dinv, b, Fin):
    def body(acc_ref, hs_ref, dinv_ref, b_ref, x_ref):
        a = acc_ref[0] + acc_ref[1] + hs_ref[...]
        dinv = dinv_ref[...]
        xv = jnp.tanh(dinv * a + b_ref[...])
        x_ref[...] = jnp.where(dinv > 0, xv, 0.0)

    return pl.pallas_call(
        body,
        grid=(NP // _BR,),
        in_specs=[
            pl.BlockSpec((NC, _BR, Fin), lambda i: (0, i, 0)),
            pl.BlockSpec((_BR, Fin), lambda i: (i, 0)),
            pl.BlockSpec((_BR, 1), lambda i: (i, 0)),
            pl.BlockSpec((1, Fin), lambda i: (0, 0)),
        ],
        out_specs=pl.BlockSpec((_BR, Fin), lambda i: (i, 0)),
        out_shape=jax.ShapeDtypeStruct((NP, Fin), jnp.float32),
    )(acc, hs, dinv, b)


def _tc_head(sel, W5p, b5, W6, b6, fc1p, fc1b, fc2, fc2b):
    def body(s_ref, w5_ref, b5_ref, w6_ref, b6_ref, f1_ref, f1b_ref,
             f2_ref, f2b_ref, o_ref):
        s = s_ref[...]
        s = jnp.concatenate(
            [s[:, :96], jnp.tanh(s[:, 96:97]), s[:, 97:]], axis=1)
        c5 = jnp.maximum(
            jnp.dot(s, w5_ref[...],
                    preferred_element_type=jnp.float32) + b5_ref[...], 0.0)
        c4 = c5.reshape(NG, 16, 2, 16)
        p = jnp.max(c4, axis=2)
        w6 = w6_ref[...]
        b6 = b6_ref[...]
        outs = []
        for l in range(11):
            a = jnp.zeros((NG, 32), jnp.float32)
            for t in range(5):
                a = a + jnp.dot(p[:, l + t, :], w6[16 * t:16 * t + 16, :],
                                preferred_element_type=jnp.float32)
            outs.append(jnp.maximum(a + b6, 0.0))
        h6 = jnp.concatenate(outs, axis=1)
        h = jnp.maximum(
            jnp.dot(h6, f1_ref[...],
                    preferred_element_type=jnp.float32) + f1b_ref[...], 0.0)
        logits = jnp.dot(h, f2_ref[...],
                         preferred_element_type=jnp.float32) + f2b_ref[...]
        m = jnp.max(logits, axis=1, keepdims=True)
        lse = jnp.log(jnp.sum(jnp.exp(logits - m), axis=1, keepdims=True)) + m
        o_ref[...] = logits - lse

    return pl.pallas_call(
        body,
        out_shape=jax.ShapeDtypeStruct((NG, 10), jnp.float32),
    )(sel, W5p, b5, W6, b6, fc1p, fc1b, fc2, fc2b)


def kernel(x, edge_index, batch, W1, b1, W2, b2, W3, b3, W4, b4,
           conv5_w, conv5_b, conv6_w, conv6_b, fc1_w, fc1_b, fc2_w, fc2_b):
    src = jnp.pad(edge_index[0], (0, EPAD - E)).reshape(NW, CPW, 128)
    dst = jnp.pad(edge_index[1], (0, EPAD - E)).reshape(NW, CPW, 128)
    x_pad = jnp.pad(x, ((0, NP - N), (0, 0)))

    deg_part, srcp, bounds = _sc_prep_kernel()(src, dst, batch)
    dinv, h1s = _tc_prep(deg_part.reshape(NC, NP, 1), x_pad, W1)

    srcp_g = srcp.reshape(NW, CPW // GRP, GRP * 128)
    dst_g = dst.reshape(NW, CPW // GRP, GRP * 128)
    _scatter32 = _make_scatter(32)
    acc1 = _scatter32(h1s, srcp_g, dst_g)
    x1, h2s = _tc_layer(acc1, h1s, dinv, b1.reshape(1, 32), W2, 32, 32)
    acc2 = _scatter32(h2s, srcp_g, dst_g)
    x2, h3s = _tc_layer(acc2, h2s, dinv, b2.reshape(1, 32), W3, 32, 32)
    acc3 = _scatter32(h3s, srcp_g, dst_g)
    W4p = jnp.pad(W4, ((0, 0), (0, 7)))
    x3, h4s = _tc_layer(acc3, h3s, dinv, b3.reshape(1, 32), W4p, 32, 8)
    acc4 = _make_scatter(8)(h4s, srcp_g, dst_g)

    xc = jnp.concatenate(
        [x1, x2, x3, jnp.zeros((NP, 32), jnp.float32)], axis=1)
    b4p16 = jnp.pad(b4, (0, 15))
    sel = _sc_sortpool_kernel()(acc4, h4s, dinv.reshape(NP), b4p16,
                                bounds, xc)

    W5p = jnp.pad(conv5_w[:, 0, :].T, ((0, 31), (0, 0)))
    W6 = conv6_w.transpose(2, 1, 0).reshape(80, 32)
    fc1p = fc1_w.reshape(32, 11, 128).transpose(1, 0, 2).reshape(352, 128)
    return _tc_head(sel, W5p, conv5_b.reshape(1, 16), W6,
                    conv6_b.reshape(1, 32), fc1p, fc1_b.reshape(1, 128),
                    fc2_w, fc2_b.reshape(1, 10))


# speedup vs baseline: 2.4026x; 1.0009x over previous
import functools

import jax
import jax.numpy as jnp
from jax import lax
from jax.experimental import pallas as pl
from jax.experimental.pallas import tpu as pltpu
from jax.experimental.pallas import tpu_sc as plsc

N = 10000
NP = 10240
E = 320000
F0 = 128
NG = 64
K = 30
NC = 2
NS = 16
NW = NC * NS
CPW = 80
EPT = CPW * 128
EPAD = NW * EPT
RPT = NP // NS
GRP = 5
DUMMY = N

def _mesh():
    return plsc.VectorSubcoreMesh(core_axis_name="c", subcore_axis_name="s",
                                  num_cores=NC, num_subcores=NS)


@functools.cache
def _sc_prep_kernel():
    return functools.partial(
        pl.kernel,
        mesh=_mesh(),
        compiler_params=pltpu.CompilerParams(needs_layout_passes=False),
        out_type=[
            jax.ShapeDtypeStruct((NC, NP), jnp.float32),
            jax.ShapeDtypeStruct((NW, CPW, 128), jnp.int32),
            jax.ShapeDtypeStruct((128,), jnp.int32),
        ],
        scratch_types=[
            pltpu.VMEM((CPW, 128), jnp.int32),
            pltpu.VMEM((CPW, 128), jnp.int32),
            pltpu.VMEM((CPW, 128), jnp.int32),
            pltpu.VMEM((NP,), jnp.float32),
            pltpu.VMEM_SHARED((NS, NP), jnp.float32),
            pltpu.VMEM((NS, RPT), jnp.float32),
            pltpu.VMEM((RPT,), jnp.float32),
            pltpu.VMEM((N,), jnp.int32),
            pltpu.VMEM((64,), jnp.int32),
            pltpu.VMEM((128,), jnp.int32),
        ],
    )(_sc_prep_body)


def _sc_prep_body(src_hbm, dst_hbm, batch_hbm, deg_out, srcp_out, bounds_out,
             src_v, dst_v, srcp_v, deg_v, shared_deg, part_v, red_v,
             batch_v, counts_v, bounds_v):
    cid = lax.axis_index("c")
    sid = lax.axis_index("s")
    wid = sid * NC + cid

    zf = jnp.zeros((16,), jnp.float32)

    def _zero(i, _):
        deg_v[pl.ds(i * 16, 16)] = zf
        return 0
    lax.fori_loop(0, NP // 16, _zero, 0)

    pltpu.sync_copy(src_hbm.at[wid], src_v)
    pltpu.sync_copy(dst_hbm.at[wid], dst_v)

    dummy16 = jnp.full((16,), DUMMY, jnp.int32)

    def _edges(j, _):
        for k in range(8):
            s = src_v[j, pl.ds(k * 16, 16)]
            d = dst_v[j, pl.ds(k * 16, 16)]
            m = s != d
            plsc.addupdate_scatter(
                deg_v, [d], jnp.where(m, 1.0, 0.0).astype(jnp.float32))
            srcp_v[j, pl.ds(k * 16, 16)] = jnp.where(m, s, dummy16)
        return 0
    lax.fori_loop(0, CPW, _edges, 0)
    pltpu.sync_copy(srcp_v, srcp_out.at[wid])

    pltpu.sync_copy(deg_v, shared_deg.at[sid])
    plsc.subcore_barrier()
    pltpu.sync_copy(shared_deg.at[:, pl.ds(sid * RPT, RPT)], part_v)

    def _red(i, _):
        acc = jnp.zeros((16,), jnp.float32)
        for k in range(NS):
            acc = acc + part_v[k, pl.ds(i * 16, 16)]
        red_v[pl.ds(i * 16, 16)] = acc
        return 0
    lax.fori_loop(0, RPT // 16, _red, 0)
    pltpu.sync_copy(red_v, deg_out.at[cid, pl.ds(sid * RPT, RPT)])

    @pl.when(jnp.logical_and(cid == 0, sid == 0))
    def _bounds():
        pltpu.sync_copy(batch_hbm, batch_v)
        zi = jnp.zeros((16,), jnp.int32)
        for i in range(4):
            counts_v[pl.ds(i * 16, 16)] = zi
        ones_i = jnp.ones((16,), jnp.int32)

        def _cnt(i, _):
            b = batch_v[pl.ds(i * 16, 16)]
            plsc.addupdate_scatter(counts_v, [b], ones_i)
            return 0
        lax.fori_loop(0, N // 16, _cnt, 0)

        carry = jnp.int32(0)
        for g in range(4):
            c = counts_v[pl.ds(g * 16, 16)]
            cs = plsc.cumsum(c)
            bounds_v[pl.ds(g * 16, 16)] = carry + cs - c
            bounds_v[pl.ds(64 + g * 16, 16)] = carry + cs
            carry = carry + jnp.sum(c)
        pltpu.sync_copy(bounds_v, bounds_out)


@functools.cache
def _make_scatter(F):
    @functools.partial(
        pl.kernel,
        mesh=_mesh(),
        compiler_params=pltpu.CompilerParams(needs_layout_passes=False,
                                             use_tc_tiling_on_sc=False),
        out_type=jax.ShapeDtypeStruct((NC, NP, F), jnp.float32),
        scratch_types=[
            pltpu.VMEM((CPW // GRP, GRP * 128), jnp.int32),
            pltpu.VMEM((CPW // GRP, GRP * 128), jnp.int32),
            pltpu.VMEM((GRP * 128, F), jnp.float32),
            pltpu.VMEM((GRP * 128, F), jnp.float32),
            pltpu.VMEM((RPT, F), jnp.float32),
            pltpu.VMEM_SHARED((NP, F), jnp.float32),
            pltpu.VMEM_SHARED((NP, F), jnp.float32),
            pltpu.SemaphoreType.DMA,
        ],
    )
    def _scatter(hs_hbm, srcp_hbm, dst_hbm, acc_out,
                 sidx, didx, rows0, rows1, zbuf, acc_sh, hs_sh, sem):
        cid = lax.axis_index("c")
        sid = lax.axis_index("s")
        wid = sid * NC + cid

        zf = jnp.zeros((16,), jnp.float32)

        def _zero(i, _):
            for k in range(F // 16):
                zbuf[i, pl.ds(k * 16, 16)] = zf
            return 0
        lax.fori_loop(0, RPT, _zero, 0)
        pltpu.sync_copy(zbuf, acc_sh.at[pl.ds(sid * RPT, RPT), :])
        pltpu.sync_copy(hs_hbm.at[pl.ds(sid * RPT, RPT), :],
                        hs_sh.at[pl.ds(sid * RPT, RPT), :])

        pltpu.sync_copy(srcp_hbm.at[wid], sidx)
        pltpu.sync_copy(dst_hbm.at[wid], didx)
        plsc.subcore_barrier()

        nt = CPW // GRP
        pltpu.async_copy(hs_sh.at[sidx.at[0]], rows0, sem)

        def _edge_pair(j2, _):
            j0 = 2 * j2
            pltpu.make_async_copy(hs_sh.at[sidx.at[j0]], rows0, sem).wait()
            pltpu.async_copy(hs_sh.at[sidx.at[j0 + 1]], rows1, sem)
            pltpu.sync_copy(rows0, acc_sh.at[didx.at[j0]], add=True)
            pltpu.make_async_copy(
                hs_sh.at[sidx.at[j0 + 1]], rows1, sem).wait()

            @pl.when(j2 + 1 < nt // 2)
            def _prefetch():
                pltpu.async_copy(hs_sh.at[sidx.at[j0 + 2]], rows0, sem)

            pltpu.sync_copy(rows1, acc_sh.at[didx.at[j0 + 1]], add=True)
            return 0
        lax.fori_loop(0, nt // 2, _edge_pair, 0)

        plsc.subcore_barrier()
        pltpu.sync_copy(acc_sh.at[pl.ds(sid * RPT, RPT), :], zbuf)
        pltpu.sync_copy(zbuf, acc_out.at[cid, pl.ds(sid * RPT, RPT), :])

    return _scatter


GPT = NG // NW
_NEG_INF = float("-inf")
_IMAX = 2147483647


@functools.cache
def _sc_sortpool_kernel():
    return functools.partial(
        pl.kernel,
        mesh=_mesh(),
        compiler_params=pltpu.CompilerParams(needs_layout_passes=False,
                                             use_tc_tiling_on_sc=False),
        out_type=jax.ShapeDtypeStruct((NG * 32, 128), jnp.float32),
        scratch_types=[
            pltpu.VMEM((RPT, 8), jnp.float32),
            pltpu.VMEM((RPT, 8), jnp.float32),
            pltpu.VMEM((RPT, 8), jnp.float32),
            pltpu.VMEM((RPT,), jnp.float32),
            pltpu.VMEM((RPT,), jnp.float32),
            pltpu.VMEM((16,), jnp.float32),
            pltpu.VMEM_SHARED((NP,), jnp.float32),
            pltpu.VMEM((NP,), jnp.float32),
            pltpu.VMEM((160,), jnp.int32),
            pltpu.VMEM((32,), jnp.int32),
            pltpu.VMEM((32, 128), jnp.float32),
            pltpu.SemaphoreType.DMA,
        ],
    )(_sc_sortpool_body)


def _sc_sortpool_body(acc4_hbm, h4s_hbm, dinv_hbm, b4_hbm, bounds_hbm, xc_hbm,
                      sel_out, a0_v, a1_v, hs_v, dv, kslice_v, b4_v,
                      keys_sh, keys_v, bounds_v, idx_buf, rows, sem):
    cid = lax.axis_index("c")
    sid = lax.axis_index("s")
    wid = sid * NC + cid
    r0 = sid * RPT

    pltpu.sync_copy(acc4_hbm.at[0, pl.ds(r0, RPT), :], a0_v)
    pltpu.sync_copy(acc4_hbm.at[1, pl.ds(r0, RPT), :], a1_v)
    pltpu.sync_copy(h4s_hbm.at[pl.ds(r0, RPT), :], hs_v)
    pltpu.sync_copy(dinv_hbm.at[pl.ds(r0, RPT)], dv)
    pltpu.sync_copy(b4_hbm, b4_v)
    pltpu.sync_copy(bounds_hbm, bounds_v.at[pl.ds(0, 128)])

    lane = jnp.arange(16, dtype=jnp.int32)
    zero16 = jnp.zeros((16,), jnp.int32)
    b4 = b4_v[...][0]

    def _keys(i, _):
        ridx = i * 16 + lane
        k0 = plsc.load_gather(a0_v, [ridx, zero16])
        k1 = plsc.load_gather(a1_v, [ridx, zero16])
        kh = plsc.load_gather(hs_v, [ridx, zero16])
        d = dv[pl.ds(i * 16, 16)]
        z = d * (k0 + k1 + kh) + b4
        kslice_v[pl.ds(i * 16, 16)] = jnp.where(d > 0, z, 0.0)
        return 0
    lax.fori_loop(0, RPT // 16, _keys, 0)

    pltpu.sync_copy(kslice_v, keys_sh.at[pl.ds(r0, RPT)])
    plsc.subcore_barrier()
    pltpu.sync_copy(keys_sh, keys_v)

    neg16 = jnp.full((16,), _NEG_INF, jnp.float32)
    dummy16 = jnp.full((16,), DUMMY, jnp.int32)

    for gi in range(GPT):
        g = wid * GPT + gi
        start = bounds_v[pl.ds(g, 16)][0]
        end = bounds_v[pl.ds(64 + g, 16)][0]
        c_lo = start // 16
        c_hi = (end + 15) // 16

        res = [dummy16, dummy16]
        resk = [jnp.zeros((16,), jnp.float32), jnp.zeros((16,), jnp.float32)]
        for r in range(K):
            def _scan(c, carry):
                m_v, i_v = carry
                base = c * 16
                kv = keys_v[pl.ds(base, 16)]
                gidx = base + lane
                valid = jnp.logical_and(gidx >= start, gidx < end)
                kv = jnp.where(valid, kv, neg16)
                upd = kv > m_v
                return jnp.where(upd, kv, m_v), jnp.where(upd, gidx, i_v)

            m_v, i_v = lax.fori_loop(
                c_lo, c_hi, _scan,
                (neg16, jnp.zeros((16,), jnp.int32)))
            m = jnp.max(m_v)
            idx = jnp.min(jnp.where(m_v == m, i_v, _IMAX))
            is_valid = m > _NEG_INF
            idx_final = jnp.where(is_valid, idx, DUMMY)
            plsc.store_scatter(
                keys_v, [jnp.full((16,), idx, jnp.int32)], neg16,
                mask=jnp.logical_and(lane == 0, is_valid))
            q, sl = divmod(r, 16)
            res[q] = jnp.where(lane == sl, idx_final, res[q])
            resk[q] = jnp.where(
                jnp.logical_and(lane == sl, is_valid), m, resk[q])

        idx_buf[pl.ds(0, 16)] = res[0]
        idx_buf[pl.ds(16, 16)] = res[1]
        pltpu.async_copy(xc_hbm.at[idx_buf], rows, sem).wait()
        for half in range(2):
            plsc.store_scatter(
                rows, [lane + 16 * half, jnp.full((16,), 96, jnp.int32)],
                resk[half])
        pltpu.sync_copy(rows, sel_out.at[pl.ds(g * 32, 32), :])


_BR = 1024


def _tc_prep(deg3, x_pad, W1):
    def body(deg_ref, x_ref, w_ref, dinv_ref, hs_ref):
        deg = deg_ref[0] + deg_ref[1]
        dinv = jnp.where(deg > 0, lax.rsqrt(deg), 0.0)
        dinv_ref[...] = dinv
        h = jnp.dot(x_ref[...], w_ref[...],
                    preferred_element_type=jnp.float32)
        hs_ref[...] = dinv * h

    return pl.pallas_call(
        body,
        grid=(NP // _BR,),
        in_specs=[
            pl.BlockSpec((NC, _BR, 1), lambda i: (0, i, 0)),
            pl.BlockSpec((_BR, F0), lambda i: (i, 0)),
            pl.BlockSpec((F0, 32), lambda i: (0, 0)),
        ],
        out_specs=[
            pl.BlockSpec((_BR, 1), lambda i: (i, 0)),
            pl.BlockSpec((_BR, 32), lambda i: (i, 0)),
        ],
        out_shape=[
            jax.ShapeDtypeStruct((NP, 1), jnp.float32),
            jax.ShapeDtypeStruct((NP, 32), jnp.float32),
        ],
    )(deg3, x_pad, W1)


def _tc_layer(acc, hs, dinv, b, Wn, Fin, Fn):
    def body(acc_ref, hs_ref, dinv_ref, b_ref, wn_ref, x_ref, hn_ref):
        a = acc_ref[0] + acc_ref[1] + hs_ref[...]
        dinv = dinv_ref[...]
        xv = jnp.tanh(dinv * a + b_ref[...])
        xv = jnp.where(dinv > 0, xv, 0.0)
        x_ref[...] = xv
        hn_ref[...] = dinv * jnp.dot(xv, wn_ref[...],
                                     preferred_element_type=jnp.float32)

    return pl.pallas_call(
        body,
        grid=(NP // _BR,),
        in_specs=[
            pl.BlockSpec((NC, _BR, Fin), lambda i: (0, i, 0)),
            pl.BlockSpec((_BR, Fin), lambda i: (i, 0)),
            pl.BlockSpec((_BR, 1), lambda i: (i, 0)),
            pl.BlockSpec((1, Fin), lambda i: (0, 0)),
            pl.BlockSpec((Fin, Fn), lambda i: (0, 0)),
        ],
        out_specs=[
            pl.BlockSpec((_BR, Fin), lambda i: (i, 0)),
            pl.BlockSpec((_BR, Fn), lambda i: (i, 0)),
        ],
        out_shape=[
            jax.ShapeDtypeStruct((NP, Fin), jnp.float32),
            jax.ShapeDtypeStruct((NP, Fn), jnp.float32),
        ],
    )(acc, hs, dinv, b, Wn)


def _tc_head(sel, W5p, b5, W6, b6, fc1p, fc1b, fc2, fc2b):
    def body(s_ref, w5_ref, b5_ref, w6_ref, b6_ref, f1_ref, f1b_ref,
             f2_ref, f2b_ref, o_ref):
        s = s_ref[...]
        s = jnp.concatenate(
            [s[:, :96], jnp.tanh(s[:, 96:97]), s[:, 97:]], axis=1)
        c5 = jnp.maximum(
            jnp.dot(s, w5_ref[...],
                    preferred_element_type=jnp.float32) + b5_ref[...], 0.0)
        c4 = c5.reshape(NG, 16, 2, 16)
        p = jnp.max(c4, axis=2)
        w6 = w6_ref[...]
        b6 = b6_ref[...]
        outs = []
        for l in range(11):
            a = jnp.zeros((NG, 32), jnp.float32)
            for t in range(5):
                a = a + jnp.dot(p[:, l + t, :], w6[16 * t:16 * t + 16, :],
                                preferred_element_type=jnp.float32)
            outs.append(jnp.maximum(a + b6, 0.0))
        h6 = jnp.concatenate(outs, axis=1)
        h = jnp.maximum(
            jnp.dot(h6, f1_ref[...],
                    preferred_element_type=jnp.float32) + f1b_ref[...], 0.0)
        logits = jnp.dot(h, f2_ref[...],
                         preferred_element_type=jnp.float32) + f2b_ref[...]
        m = jnp.max(logits, axis=1, keepdims=True)
        lse = jnp.log(jnp.sum(jnp.exp(logits - m), axis=1, keepdims=True)) + m
        o_ref[...] = logits - lse

    return pl.pallas_call(
        body,
        out_shape=jax.ShapeDtypeStruct((NG, 10), jnp.float32),
    )(sel, W5p, b5, W6, b6, fc1p, fc1b, fc2, fc2b)


def kernel(x, edge_index, batch, W1, b1, W2, b2, W3, b3, W4, b4,
           conv5_w, conv5_b, conv6_w, conv6_b, fc1_w, fc1_b, fc2_w, fc2_b):
    src = jnp.pad(edge_index[0], (0, EPAD - E)).reshape(NW, CPW, 128)
    dst = jnp.pad(edge_index[1], (0, EPAD - E)).reshape(NW, CPW, 128)
    x_pad = jnp.pad(x, ((0, NP - N), (0, 0)))

    deg_part, srcp, bounds = _sc_prep_kernel()(src, dst, batch)
    dinv, h1s = _tc_prep(deg_part.reshape(NC, NP, 1), x_pad, W1)

    srcp_g = srcp.reshape(NW, CPW // GRP, GRP * 128)
    dst_g = dst.reshape(NW, CPW // GRP, GRP * 128)
    _scatter32 = _make_scatter(32)
    acc1 = _scatter32(h1s, srcp_g, dst_g)
    x1, h2s = _tc_layer(acc1, h1s, dinv, b1.reshape(1, 32), W2, 32, 32)
    acc2 = _scatter32(h2s, srcp_g, dst_g)
    x2, h3s = _tc_layer(acc2, h2s, dinv, b2.reshape(1, 32), W3, 32, 32)
    acc3 = _scatter32(h3s, srcp_g, dst_g)
    W4p = jnp.pad(W4, ((0, 0), (0, 7)))
    x3, h4s = _tc_layer(acc3, h3s, dinv, b3.reshape(1, 32), W4p, 32, 8)
    acc4 = _make_scatter(8)(h4s, srcp_g, dst_g)

    xc = jnp.concatenate(
        [x1, x2, x3, jnp.zeros((NP, 32), jnp.float32)], axis=1)
    b4p16 = jnp.pad(b4, (0, 15))
    sel = _sc_sortpool_kernel()(acc4, h4s, dinv.reshape(NP), b4p16,
                                bounds, xc)

    W5p = jnp.pad(conv5_w[:, 0, :].T, ((0, 31), (0, 0)))
    W6 = conv6_w.transpose(2, 1, 0).reshape(80, 32)
    fc1p = fc1_w.reshape(32, 11, 128).transpose(1, 0, 2).reshape(352, 128)
    return _tc_head(sel, W5p, conv5_b.reshape(1, 16), W6,
                    conv6_b.reshape(1, 32), fc1p, fc1_b.reshape(1, 128),
                    fc2_w, fc2_b.reshape(1, 10))
